# Initial kernel scaffold; baseline (speedup 1.0000x reference)
import jax
import jax.numpy as jnp
from jax.experimental import pallas as pl


def kernel(x, vec, edge_index, r_ij, f_ij, d_ij, g_ln, b_ln, Wq, bq, Wk, bk, Wv, bv, Wvec, Wdk, bdk, Wdv, bdv, Wo, bo):
    raise NotImplementedError("write your pallas kernel here")


# speedup vs baseline: 21.2803x; 21.2803x over previous
import functools
import math

import jax
import jax.numpy as jnp
from jax import lax
from jax.experimental import pallas as pl
from jax.experimental.pallas import tpu as pltpu
from jax.experimental.pallas import tpu_sc as plsc

N = 10000
D = 128
H = 8
DH = 16
E = 160000
R = 64
CUT = 5.0

NC = 2
NS = 16
NW = NC * NS


BN1 = 1000


def _silu(a):
    return a * jax.nn.sigmoid(a)


def _node_body(x_ref, vec_ref, gln_ref, bln_ref, wq_ref, bq_ref, wk_ref,
               bk_ref, wv_ref, bv_ref, wvec_ref,
               q_ref, kt_ref, vd_ref, v3_ref):
    x = x_ref[...]
    mu = jnp.mean(x, axis=-1, keepdims=True)
    xc = x - mu
    var = jnp.mean(xc * xc, axis=-1, keepdims=True)
    xn = xc * lax.rsqrt(var + 1e-5) * gln_ref[...] + bln_ref[...]
    dot = lambda a, w: jnp.dot(a, w, preferred_element_type=jnp.float32)
    q_ref[...] = dot(xn, wq_ref[...]) + bq_ref[...]
    kt_ref[:, 0:D] = dot(xn, wk_ref[...]) + bk_ref[...]
    kt_ref[:, D:4 * D] = dot(xn, wv_ref[...]) + bv_ref[...]
    vd = jnp.zeros((x.shape[0], D), jnp.float32)
    for c in range(3):
        vc = vec_ref[:, c, :]
        vp = dot(vc, wvec_ref[...])
        vd = vd + vp[:, :D] * vp[:, D:2 * D]
        v3_ref[:, c, :] = vp[:, 2 * D:3 * D]
        kt_ref[:, 4 * D + c * D:4 * D + (c + 1) * D] = vc
    vd_ref[...] = vd


def _node_stage(x, vec, g_ln2, b_ln2, Wq, bq2, Wk, bk2, Wv_p, bv_p2, Wvec):
    def full(shape):
        return pl.BlockSpec(shape, lambda i: tuple(0 for _ in shape))
    return pl.pallas_call(
        _node_body,
        grid=(N // BN1,),
        in_specs=[
            pl.BlockSpec((BN1, D), lambda i: (i, 0)),
            pl.BlockSpec((BN1, 3, D), lambda i: (i, 0, 0)),
            full((1, D)), full((1, D)),
            full((D, D)), full((1, D)),
            full((D, D)), full((1, D)),
            full((D, 3 * D)), full((1, 3 * D)),
            full((D, 3 * D)),
        ],
        out_specs=[
            pl.BlockSpec((BN1, D), lambda i: (i, 0)),
            pl.BlockSpec((BN1, 7 * D), lambda i: (i, 0)),
            pl.BlockSpec((BN1, D), lambda i: (i, 0)),
            pl.BlockSpec((BN1, 3, D), lambda i: (i, 0, 0)),
        ],
        out_shape=[
            jax.ShapeDtypeStruct((N, D), jnp.float32),
            jax.ShapeDtypeStruct((N, 7 * D), jnp.float32),
            jax.ShapeDtypeStruct((N, D), jnp.float32),
            jax.ShapeDtypeStruct((N, 3, D), jnp.float32),
        ],
    )(x, vec, g_ln2, b_ln2, Wq, bq2, Wk, bk2, Wv_p, bv_p2, Wvec)


EPW = E // NW
CHG = 40
NCHG = EPW // CHG


def _sc_gather(q, kt, src, dst):
    mesh = plsc.VectorSubcoreMesh(core_axis_name="c", subcore_axis_name="s")

    @functools.partial(
        pl.kernel,
        mesh=mesh,
        out_type=[jax.ShapeDtypeStruct((E, D), jnp.float32),
                  jax.ShapeDtypeStruct((E, 7 * D), jnp.float32)],
        scratch_types=[
            pltpu.VMEM((CHG,), jnp.int32),
            pltpu.VMEM((CHG,), jnp.int32),
            pltpu.VMEM((CHG, D), jnp.float32),
            pltpu.VMEM((CHG, 7 * D), jnp.float32),
            pltpu.SemaphoreType.DMA,
            pltpu.SemaphoreType.DMA,
        ],
    )
    def gk(q_hbm, kt_hbm, src_hbm, dst_hbm, qd_out, g_out,
           didx, sidx, qrows, krows, sem1, sem2):
        wid = lax.axis_index("s") * NC + lax.axis_index("c")
        base0 = wid * EPW

        def body(i, carry):
            base = base0 + i * CHG
            pltpu.sync_copy(dst_hbm.at[pl.ds(base, CHG)], didx)
            pltpu.sync_copy(src_hbm.at[pl.ds(base, CHG)], sidx)
            cp1 = pltpu.async_copy(q_hbm.at[didx], qrows, sem1)
            cp2 = pltpu.async_copy(kt_hbm.at[sidx], krows, sem2)
            cp1.wait()
            cp2.wait()
            pltpu.sync_copy(qrows, qd_out.at[pl.ds(base, CHG)])
            pltpu.sync_copy(krows, g_out.at[pl.ds(base, CHG)])
            return carry

        lax.fori_loop(0, NCHG, body, 0)

    return gk(q, kt, src, dst)


BE = 1000


def _edge_body(qd_ref, g_ref, f_ref, r_ref, d0_ref, d1_ref, d2_ref,
               wdk_ref, bdk_ref, wdv_ref, bdv_ref, m_ref,
               xe_ref, vm0_ref, vm1_ref, vm2_ref):
    dot = lambda a, w: jnp.dot(a, w, preferred_element_type=jnp.float32)
    f = f_ref[...]
    dk = _silu(dot(f, wdk_ref[...]) + bdk_ref[...])
    dvp = _silu(dot(f, wdv_ref[...]) + bdv_ref[...])
    attn_pre = dot(qd_ref[...] * g_ref[:, :D] * dk, m_ref[...])
    r = r_ref[...]
    cut = 0.5 * (jnp.cos(r * (math.pi / CUT)) + 1.0) * (r < CUT).astype(jnp.float32)
    a = _silu(attn_pre) * cut
    vjx = g_ref[:, D:2 * D] * dvp[:, :D]
    v1e = g_ref[:, 2 * D:3 * D] * dvp[:, D:2 * D]
    v2e = g_ref[:, 3 * D:4 * D] * dvp[:, 2 * D:3 * D]
    xe_ref[...] = vjx * a
    for c, (dref, oref) in enumerate(((d0_ref, vm0_ref), (d1_ref, vm1_ref),
                                      (d2_ref, vm2_ref))):
        oref[...] = g_ref[:, 4 * D + c * D:4 * D + (c + 1) * D] * v1e + v2e * dref[...]


def _edge_stage(qd, g, f_ij, r2, d0, d1, d2, Wdk, bdk2, Wdv_p, bdv_p2, m):
    def full(shape):
        return pl.BlockSpec(shape, lambda i: tuple(0 for _ in shape))
    row = lambda w: pl.BlockSpec((BE, w), lambda i: (i, 0))
    return pl.pallas_call(
        _edge_body,
        grid=(E // BE,),
        in_specs=[
            row(D), row(7 * D), row(R), row(1), row(1), row(1), row(1),
            full((R, D)), full((1, D)),
            full((R, 3 * D)), full((1, 3 * D)),
            full((D, D)),
        ],
        out_specs=[row(D), row(D), row(D), row(D)],
        out_shape=[jax.ShapeDtypeStruct((E, D), jnp.float32)] * 4,
    )(qd, g, f_ij, r2, d0, d1, d2, Wdk, bdk2, Wdv_p, bdv_p2, m)


EPT = E // NS
CHS = 40
NCHS = EPT // CHS
NPAD = 10240
NPT = NPAD // NS


def _sc_scatter(m0, m1, m2, m3, dst, zrows):
    mesh = plsc.VectorSubcoreMesh(core_axis_name="c", subcore_axis_name="s")

    @functools.partial(
        pl.kernel,
        mesh=mesh,
        out_type=[jax.ShapeDtypeStruct((NPAD, D), jnp.float32)] * 4,
        scratch_types=[
            pltpu.VMEM((CHS,), jnp.int32),
            pltpu.VMEM((CHS, D), jnp.float32),
            pltpu.VMEM_SHARED((NPAD, D), jnp.float32),
        ],
    )
    def sk(m0_hbm, m1_hbm, m2_hbm, m3_hbm, dst_hbm, z_hbm,
           o0, o1, o2, o3, idx, rows, acc):
        cid = lax.axis_index("c")
        sid = lax.axis_index("s")
        row0 = sid * NPT
        planes = ((m0_hbm, o0), (m1_hbm, o1), (m2_hbm, o2), (m3_hbm, o3))
        for plane, (m_hbm, o_hbm) in enumerate(planes):
            @pl.when(cid == (plane // 2))
            def _():
                pltpu.sync_copy(z_hbm.at[pl.ds(row0, NPT)],
                                acc.at[pl.ds(row0, NPT)])
                plsc.subcore_barrier()

                def body(i, carry):
                    base = sid * EPT + i * CHS
                    pltpu.sync_copy(dst_hbm.at[pl.ds(base, CHS)], idx)
                    pltpu.sync_copy(m_hbm.at[pl.ds(base, CHS)], rows)
                    pltpu.sync_copy(rows, acc.at[idx], add=True)
                    return carry

                lax.fori_loop(0, NCHS, body, 0)
                plsc.subcore_barrier()
                pltpu.sync_copy(acc.at[pl.ds(row0, NPT)],
                                o_hbm.at[pl.ds(row0, NPT)])
                plsc.subcore_barrier()

    return sk(m0, m1, m2, m3, dst, zrows)


BN3 = 2000


def _final_body(xa_ref, a0_ref, a1_ref, a2_ref, vd_ref, v3_ref,
                wo_ref, bo_ref, dx_ref, dvec_ref):
    o = jnp.dot(xa_ref[...], wo_ref[...], preferred_element_type=jnp.float32) + bo_ref[...]
    o1 = o[:, :D]
    o2 = o[:, D:2 * D]
    o3 = o[:, 2 * D:3 * D]
    dx_ref[...] = vd_ref[...] * o2 + o3
    for c, aref in enumerate((a0_ref, a1_ref, a2_ref)):
        dvec_ref[:, c, :] = v3_ref[:, c, :] * o1 + aref[...]


def _final_stage(xagg, a0, a1, a2, vd, v3, Wo, bo2):
    def full(shape):
        return pl.BlockSpec(shape, lambda i: tuple(0 for _ in shape))
    row = lambda w: pl.BlockSpec((BN3, w), lambda i: (i, 0))
    return pl.pallas_call(
        _final_body,
        grid=(N // BN3,),
        in_specs=[
            row(D), row(D), row(D), row(D), row(D),
            pl.BlockSpec((BN3, 3, D), lambda i: (i, 0, 0)),
            full((D, 3 * D)), full((1, 3 * D)),
        ],
        out_specs=[
            row(D),
            pl.BlockSpec((BN3, 3, D), lambda i: (i, 0, 0)),
        ],
        out_shape=[
            jax.ShapeDtypeStruct((N, D), jnp.float32),
            jax.ShapeDtypeStruct((N, 3, D), jnp.float32),
        ],
    )(xagg, a0, a1, a2, vd, v3, Wo, bo2)


def kernel(x, vec, edge_index, r_ij, f_ij, d_ij, g_ln, b_ln, Wq, bq, Wk, bk,
           Wv, bv, Wvec, Wdk, bdk, Wdv, bdv, Wo, bo):
    f32 = jnp.float32
    Wv_p = Wv.reshape(D, H, 3, DH).transpose(0, 2, 1, 3).reshape(D, 3 * D)
    bv_p = bv.reshape(H, 3, DH).transpose(1, 0, 2).reshape(3 * D)
    Wdv_p = Wdv.reshape(R, H, 3, DH).transpose(0, 2, 1, 3).reshape(R, 3 * D)
    bdv_p = bdv.reshape(H, 3, DH).transpose(1, 0, 2).reshape(3 * D)
    src = edge_index[0]
    dst = edge_index[1]
    r2 = r_ij.reshape(E, 1)
    d0 = d_ij[:, 0:1]
    d1 = d_ij[:, 1:2]
    d2 = d_ij[:, 2:3]
    m = jnp.kron(jnp.eye(H, dtype=f32), jnp.ones((DH, DH), f32))
    zrows = jnp.zeros((NPAD, D), f32)

    q, kt, vd, v3 = _node_stage(
        x, vec, g_ln.reshape(1, D), b_ln.reshape(1, D),
        Wq, bq.reshape(1, D), Wk, bk.reshape(1, D),
        Wv_p, bv_p.reshape(1, 3 * D), Wvec)
    qd, g = _sc_gather(q, kt, src, dst)
    xe, vm0, vm1, vm2 = _edge_stage(
        qd, g, f_ij, r2, d0, d1, d2,
        Wdk, bdk.reshape(1, D), Wdv_p, bdv_p.reshape(1, 3 * D), m)
    xagg, a0, a1, a2 = _sc_scatter(xe, vm0, vm1, vm2, dst, zrows)
    dx, dvec = _final_stage(xagg[:N], a0[:N], a1[:N], a2[:N], vd, v3,
                            Wo, bo.reshape(1, 3 * D))
    return dx, dvec


# speedup vs baseline: 29.9461x; 1.4072x over previous
import functools
import math

import jax
import jax.numpy as jnp
from jax import lax
from jax.experimental import pallas as pl
from jax.experimental.pallas import tpu as pltpu
from jax.experimental.pallas import tpu_sc as plsc

N = 10000
D = 128
H = 8
DH = 16
E = 160000
R = 64
CUT = 5.0

NC = 2
NS = 16
NW = NC * NS


BN1 = 1000


def _silu(a):
    return a / (1.0 + jnp.exp(-a))


def _node_body(x_ref, vec_ref, gln_ref, bln_ref, wq_ref, bq_ref, wk_ref,
               bk_ref, wv_ref, bv_ref, wvec_ref,
               q_ref, kt_ref, vd_ref, v3_ref):
    x = x_ref[...]
    mu = jnp.mean(x, axis=-1, keepdims=True)
    xc = x - mu
    var = jnp.mean(xc * xc, axis=-1, keepdims=True)
    xn = xc * lax.rsqrt(var + 1e-5) * gln_ref[...] + bln_ref[...]
    dot = lambda a, w: jnp.dot(a, w, preferred_element_type=jnp.float32)
    q_ref[...] = dot(xn, wq_ref[...]) + bq_ref[...]
    kt_ref[:, 0:D] = dot(xn, wk_ref[...]) + bk_ref[...]
    kt_ref[:, D:4 * D] = dot(xn, wv_ref[...]) + bv_ref[...]
    vd = jnp.zeros((x.shape[0], D), jnp.float32)
    for c in range(3):
        vc = vec_ref[:, c, :]
        vp = dot(vc, wvec_ref[...])
        vd = vd + vp[:, :D] * vp[:, D:2 * D]
        v3_ref[:, c, :] = vp[:, 2 * D:3 * D]
        kt_ref[:, 4 * D + c * D:4 * D + (c + 1) * D] = vc
    vd_ref[...] = vd


def _node_stage(x, vec, g_ln2, b_ln2, Wq, bq2, Wk, bk2, Wv_p, bv_p2, Wvec):
    def full(shape):
        return pl.BlockSpec(shape, lambda i: tuple(0 for _ in shape))
    return pl.pallas_call(
        _node_body,
        grid=(N // BN1,),
        in_specs=[
            pl.BlockSpec((BN1, D), lambda i: (i, 0)),
            pl.BlockSpec((BN1, 3, D), lambda i: (i, 0, 0)),
            full((1, D)), full((1, D)),
            full((D, D)), full((1, D)),
            full((D, D)), full((1, D)),
            full((D, 3 * D)), full((1, 3 * D)),
            full((D, 3 * D)),
        ],
        out_specs=[
            pl.BlockSpec((BN1, D), lambda i: (i, 0)),
            pl.BlockSpec((BN1, 7 * D), lambda i: (i, 0)),
            pl.BlockSpec((BN1, D), lambda i: (i, 0)),
            pl.BlockSpec((BN1, 3, D), lambda i: (i, 0, 0)),
        ],
        out_shape=[
            jax.ShapeDtypeStruct((N, D), jnp.float32),
            jax.ShapeDtypeStruct((N, 7 * D), jnp.float32),
            jax.ShapeDtypeStruct((N, D), jnp.float32),
            jax.ShapeDtypeStruct((N, 3, D), jnp.float32),
        ],
    )(x, vec, g_ln2, b_ln2, Wq, bq2, Wk, bk2, Wv_p, bv_p2, Wvec)


EPW = E // NW
CHG = 40
NCHG = EPW // CHG


def _sc_gather(q, kt, src3, dst3):
    mesh = plsc.VectorSubcoreMesh(core_axis_name="c", subcore_axis_name="s")

    @functools.partial(
        pl.kernel,
        mesh=mesh,
        out_type=[jax.ShapeDtypeStruct((E, D), jnp.float32),
                  jax.ShapeDtypeStruct((E, 7 * D), jnp.float32)],
        scratch_types=[
            pltpu.VMEM((NCHG, CHG), jnp.int32),
            pltpu.VMEM((NCHG, CHG), jnp.int32),
            [pltpu.VMEM((CHG, D), jnp.float32)] * 2,
            [pltpu.VMEM((CHG, 7 * D), jnp.float32)] * 2,
            [pltpu.SemaphoreType.DMA] * 8,
        ],
    )
    def gk(q_hbm, kt_hbm, src_hbm, dst_hbm, qd_out, g_out,
           didx, sidx, qbuf, kbuf, sems):
        gq = sems[0:2]
        gk_ = sems[2:4]
        wq = sems[4:6]
        wk = sems[6:8]
        wid = lax.axis_index("s") * NC + lax.axis_index("c")
        base0 = wid * EPW
        pltpu.sync_copy(dst_hbm.at[wid], didx)
        pltpu.sync_copy(src_hbm.at[wid], sidx)

        def fire_gather(i, b):
            pltpu.async_copy(q_hbm.at[didx.at[i]], qbuf[b], gq[b])
            pltpu.async_copy(kt_hbm.at[sidx.at[i]], kbuf[b], gk_[b])

        def wait_gather(i, b):
            pltpu.make_async_copy(q_hbm.at[didx.at[i]], qbuf[b], gq[b]).wait()
            pltpu.make_async_copy(kt_hbm.at[sidx.at[i]], kbuf[b], gk_[b]).wait()

        def fire_write(i, b):
            base = base0 + i * CHG
            pltpu.async_copy(qbuf[b], qd_out.at[pl.ds(base, CHG)], wq[b])
            pltpu.async_copy(kbuf[b], g_out.at[pl.ds(base, CHG)], wk[b])

        def wait_write(i, b):
            base = base0 + i * CHG
            pltpu.make_async_copy(qbuf[b], qd_out.at[pl.ds(base, CHG)], wq[b]).wait()
            pltpu.make_async_copy(kbuf[b], g_out.at[pl.ds(base, CHG)], wk[b]).wait()

        fire_gather(0, 0)
        fire_gather(1, 1)

        def pair(j, carry):
            i0 = 2 * j
            for b in range(2):
                i = i0 + b
                wait_gather(i, b)
                fire_write(i, b)
            for b in range(2):
                i = i0 + b
                wait_write(i, b)

                @pl.when(i + 2 < NCHG)
                def _():
                    fire_gather(i + 2, b)
            return carry

        lax.fori_loop(0, NCHG // 2, pair, 0)
        last = NCHG - 1
        wait_gather(last, 0)
        fire_write(last, 0)
        wait_write(last, 0)

    return gk(q, kt, src3, dst3)


def _cut_body(r_ref, cut_ref):
    r = r_ref[...]
    cut_ref[...] = (0.5 * (jnp.cos(r * (math.pi / CUT)) + 1.0)
                    * (r < CUT).astype(jnp.float32))


def _cut_stage(rmat):
    return pl.pallas_call(
        _cut_body,
        out_shape=jax.ShapeDtypeStruct((E // D, D), jnp.float32),
    )(rmat)


BE = 1000


def _edge_body(qd_ref, g_ref, f_ref, cut_ref, d0_ref, d1_ref, d2_ref,
               wdk_ref, bdk_ref, wdv_ref, bdv_ref, m_ref,
               xe_ref, vm0_ref, vm1_ref, vm2_ref):
    dot = lambda a, w: jnp.dot(a, w, preferred_element_type=jnp.float32)
    f = f_ref[...]
    dk = _silu(dot(f, wdk_ref[...]) + bdk_ref[...])
    dvp = _silu(dot(f, wdv_ref[...]) + bdv_ref[...])
    attn_pre = dot(qd_ref[...] * g_ref[:, :D] * dk, m_ref[...])
    a = _silu(attn_pre) * cut_ref[...]
    vjx = g_ref[:, D:2 * D] * dvp[:, :D]
    v1e = g_ref[:, 2 * D:3 * D] * dvp[:, D:2 * D]
    v2e = g_ref[:, 3 * D:4 * D] * dvp[:, 2 * D:3 * D]
    xe_ref[...] = vjx * a
    for c, (dref, oref) in enumerate(((d0_ref, vm0_ref), (d1_ref, vm1_ref),
                                      (d2_ref, vm2_ref))):
        oref[...] = g_ref[:, 4 * D + c * D:4 * D + (c + 1) * D] * v1e + v2e * dref[...]


def _edge_stage(qd, g, f_ij, cut2, d0, d1, d2, Wdk, bdk2, Wdv_p, bdv_p2, m):
    def full(shape):
        return pl.BlockSpec(shape, lambda i: tuple(0 for _ in shape))
    row = lambda w: pl.BlockSpec((BE, w), lambda i: (i, 0))
    return pl.pallas_call(
        _edge_body,
        grid=(E // BE,),
        in_specs=[
            row(D), row(7 * D), row(R), row(1), row(1), row(1), row(1),
            full((R, D)), full((1, D)),
            full((R, 3 * D)), full((1, 3 * D)),
            full((D, D)),
        ],
        out_specs=[row(D), row(D), row(D), row(D)],
        out_shape=[jax.ShapeDtypeStruct((E, D), jnp.float32)] * 4,
    )(qd, g, f_ij, cut2, d0, d1, d2, Wdk, bdk2, Wdv_p, bdv_p2, m)


EPT = E // NS
CHS = 40
NCHS = EPT // CHS
NPAD = 10240
NPT = NPAD // NS


def _sc_scatter(m0, m1, m2, m3, dst3, zrows):
    mesh = plsc.VectorSubcoreMesh(core_axis_name="c", subcore_axis_name="s")

    @functools.partial(
        pl.kernel,
        mesh=mesh,
        out_type=[jax.ShapeDtypeStruct((NPAD, D), jnp.float32)] * 4,
        scratch_types=[
            pltpu.VMEM((NCHS, CHS), jnp.int32),
            [pltpu.VMEM((CHS, D), jnp.float32)] * 2,
            pltpu.VMEM_SHARED((NPAD, D), jnp.float32),
            [pltpu.SemaphoreType.DMA] * 2,
        ],
    )
    def sk(m0_hbm, m1_hbm, m2_hbm, m3_hbm, dst_hbm, z_hbm,
           o0, o1, o2, o3, idx, rows, acc, sems):
        cid = lax.axis_index("c")
        sid = lax.axis_index("s")
        row0 = sid * NPT
        pltpu.sync_copy(dst_hbm.at[sid], idx)
        planes = ((m0_hbm, o0), (m1_hbm, o1), (m2_hbm, o2), (m3_hbm, o3))
        for plane, (m_hbm, o_hbm) in enumerate(planes):
            @pl.when(cid == (plane // 2))
            def _(m_hbm=m_hbm, o_hbm=o_hbm):
                pltpu.sync_copy(z_hbm.at[pl.ds(row0, NPT)],
                                acc.at[pl.ds(row0, NPT)])

                def fire_read(i, b):
                    base = sid * EPT + i * CHS
                    pltpu.async_copy(m_hbm.at[pl.ds(base, CHS)],
                                     rows[b], sems[b])

                def wait_read(i, b):
                    base = sid * EPT + i * CHS
                    pltpu.make_async_copy(m_hbm.at[pl.ds(base, CHS)],
                                          rows[b], sems[b]).wait()

                plsc.subcore_barrier()
                fire_read(0, 0)
                fire_read(1, 1)

                def pair(j, carry):
                    i0 = 2 * j
                    for b in range(2):
                        i = i0 + b
                        wait_read(i, b)
                        pltpu.sync_copy(rows[b], acc.at[idx.at[i]], add=True)

                        @pl.when(i + 2 < NCHS)
                        def _():
                            fire_read(i + 2, b)
                    return carry

                lax.fori_loop(0, NCHS // 2, pair, 0)
                plsc.subcore_barrier()
                pltpu.sync_copy(acc.at[pl.ds(row0, NPT)],
                                o_hbm.at[pl.ds(row0, NPT)])
                plsc.subcore_barrier()

    return sk(m0, m1, m2, m3, dst3, zrows)


BN3 = 2000


def _final_body(xa_ref, a0_ref, a1_ref, a2_ref, vd_ref, v3_ref,
                wo_ref, bo_ref, dx_ref, dvec_ref):
    o = jnp.dot(xa_ref[...], wo_ref[...], preferred_element_type=jnp.float32) + bo_ref[...]
    o1 = o[:, :D]
    o2 = o[:, D:2 * D]
    o3 = o[:, 2 * D:3 * D]
    dx_ref[...] = vd_ref[...] * o2 + o3
    for c, aref in enumerate((a0_ref, a1_ref, a2_ref)):
        dvec_ref[:, c, :] = v3_ref[:, c, :] * o1 + aref[...]


def _final_stage(xagg, a0, a1, a2, vd, v3, Wo, bo2):
    def full(shape):
        return pl.BlockSpec(shape, lambda i: tuple(0 for _ in shape))
    row = lambda w: pl.BlockSpec((BN3, w), lambda i: (i, 0))
    return pl.pallas_call(
        _final_body,
        grid=(N // BN3,),
        in_specs=[
            row(D), row(D), row(D), row(D), row(D),
            pl.BlockSpec((BN3, 3, D), lambda i: (i, 0, 0)),
            full((D, 3 * D)), full((1, 3 * D)),
        ],
        out_specs=[
            row(D),
            pl.BlockSpec((BN3, 3, D), lambda i: (i, 0, 0)),
        ],
        out_shape=[
            jax.ShapeDtypeStruct((N, D), jnp.float32),
            jax.ShapeDtypeStruct((N, 3, D), jnp.float32),
        ],
    )(xagg, a0, a1, a2, vd, v3, Wo, bo2)


def kernel(x, vec, edge_index, r_ij, f_ij, d_ij, g_ln, b_ln, Wq, bq, Wk, bk,
           Wv, bv, Wvec, Wdk, bdk, Wdv, bdv, Wo, bo):
    f32 = jnp.float32
    Wv_p = Wv.reshape(D, H, 3, DH).transpose(0, 2, 1, 3).reshape(D, 3 * D)
    bv_p = bv.reshape(H, 3, DH).transpose(1, 0, 2).reshape(3 * D)
    Wdv_p = Wdv.reshape(R, H, 3, DH).transpose(0, 2, 1, 3).reshape(R, 3 * D)
    bdv_p = bdv.reshape(H, 3, DH).transpose(1, 0, 2).reshape(3 * D)
    src = edge_index[0]
    dst = edge_index[1]
    cut2 = _cut_stage(r_ij.reshape(E // D, D)).reshape(E, 1)
    d0 = d_ij[:, 0:1]
    d1 = d_ij[:, 1:2]
    d2 = d_ij[:, 2:3]
    m = jnp.kron(jnp.eye(H, dtype=f32), jnp.ones((DH, DH), f32))
    zrows = jnp.zeros((NPAD, D), f32)

    q, kt, vd, v3 = _node_stage(
        x, vec, g_ln.reshape(1, D), b_ln.reshape(1, D),
        Wq, bq.reshape(1, D), Wk, bk.reshape(1, D),
        Wv_p, bv_p.reshape(1, 3 * D), Wvec)
    qd, g = _sc_gather(q, kt, src.reshape(NW, NCHG, CHG),
                       dst.reshape(NW, NCHG, CHG))
    xe, vm0, vm1, vm2 = _edge_stage(
        qd, g, f_ij, cut2, d0, d1, d2,
        Wdk, bdk.reshape(1, D), Wdv_p, bdv_p.reshape(1, 3 * D), m)
    xagg, a0, a1, a2 = _sc_scatter(xe, vm0, vm1, vm2,
                                   dst.reshape(NS, NCHS, CHS), zrows)
    dx, dvec = _final_stage(xagg[:N], a0[:N], a1[:N], a2[:N], vd, v3,
                            Wo, bo.reshape(1, 3 * D))
    return dx, dvec


# speedup vs baseline: 31.6871x; 1.0581x over previous
import functools
import math

import jax
import jax.numpy as jnp
from jax import lax
from jax.experimental import pallas as pl
from jax.experimental.pallas import tpu as pltpu
from jax.experimental.pallas import tpu_sc as plsc

N = 10000
D = 128
H = 8
DH = 16
E = 160000
R = 64
CUT = 5.0

NC = 2
NS = 16
NW = NC * NS


BN1 = 1000


def _silu(a):
    return a / (1.0 + jnp.exp(-a))


def _node_body(x_ref, vec_ref, gln_ref, bln_ref, wq_ref, bq_ref, wk_ref,
               bk_ref, wv_ref, bv_ref, wvec_ref,
               q_ref, kt_ref, vd_ref, v3_ref):
    x = x_ref[...]
    mu = jnp.mean(x, axis=-1, keepdims=True)
    xc = x - mu
    var = jnp.mean(xc * xc, axis=-1, keepdims=True)
    xn = xc * lax.rsqrt(var + 1e-5) * gln_ref[...] + bln_ref[...]
    dot = lambda a, w: jnp.dot(a, w, preferred_element_type=jnp.float32)
    q_ref[...] = dot(xn, wq_ref[...]) + bq_ref[...]
    kt_ref[:, 0:D] = dot(xn, wk_ref[...]) + bk_ref[...]
    kt_ref[:, D:4 * D] = dot(xn, wv_ref[...]) + bv_ref[...]
    vd = jnp.zeros((x.shape[0], D), jnp.float32)
    for c in range(3):
        vc = vec_ref[:, c, :]
        vp = dot(vc, wvec_ref[...])
        vd = vd + vp[:, :D] * vp[:, D:2 * D]
        v3_ref[:, c, :] = vp[:, 2 * D:3 * D]
        kt_ref[:, 4 * D + c * D:4 * D + (c + 1) * D] = vc
    vd_ref[...] = vd


def _node_stage(x, vec, g_ln2, b_ln2, Wq, bq2, Wk, bk2, Wv_p, bv_p2, Wvec):
    def full(shape):
        return pl.BlockSpec(shape, lambda i: tuple(0 for _ in shape))
    return pl.pallas_call(
        _node_body,
        grid=(N // BN1,),
        in_specs=[
            pl.BlockSpec((BN1, D), lambda i: (i, 0)),
            pl.BlockSpec((BN1, 3, D), lambda i: (i, 0, 0)),
            full((1, D)), full((1, D)),
            full((D, D)), full((1, D)),
            full((D, D)), full((1, D)),
            full((D, 3 * D)), full((1, 3 * D)),
            full((D, 3 * D)),
        ],
        out_specs=[
            pl.BlockSpec((BN1, D), lambda i: (i, 0)),
            pl.BlockSpec((BN1, 7 * D), lambda i: (i, 0)),
            pl.BlockSpec((BN1, D), lambda i: (i, 0)),
            pl.BlockSpec((BN1, 3, D), lambda i: (i, 0, 0)),
        ],
        out_shape=[
            jax.ShapeDtypeStruct((N, D), jnp.float32),
            jax.ShapeDtypeStruct((N, 7 * D), jnp.float32),
            jax.ShapeDtypeStruct((N, D), jnp.float32),
            jax.ShapeDtypeStruct((N, 3, D), jnp.float32),
        ],
    )(x, vec, g_ln2, b_ln2, Wq, bq2, Wk, bk2, Wv_p, bv_p2, Wvec)


EPW = E // NW
CHG = 40
NCHG = EPW // CHG


def _sc_gather(q, kt, src3, dst3):
    mesh = plsc.VectorSubcoreMesh(core_axis_name="c", subcore_axis_name="s")

    @functools.partial(
        pl.kernel,
        mesh=mesh,
        out_type=[jax.ShapeDtypeStruct((E, D), jnp.float32),
                  jax.ShapeDtypeStruct((E, 7 * D), jnp.float32)],
        scratch_types=[
            pltpu.VMEM((NCHG, CHG), jnp.int32),
            pltpu.VMEM((NCHG, CHG), jnp.int32),
            [pltpu.VMEM((CHG, D), jnp.float32)] * 2,
            [pltpu.VMEM((CHG, 7 * D), jnp.float32)] * 2,
            [pltpu.SemaphoreType.DMA] * 8,
        ],
    )
    def gk(q_hbm, kt_hbm, src_hbm, dst_hbm, qd_out, g_out,
           didx, sidx, qbuf, kbuf, sems):
        gq = sems[0:2]
        gk_ = sems[2:4]
        wq = sems[4:6]
        wk = sems[6:8]
        wid = lax.axis_index("s") * NC + lax.axis_index("c")
        base0 = wid * EPW
        pltpu.sync_copy(dst_hbm.at[wid], didx)
        pltpu.sync_copy(src_hbm.at[wid], sidx)

        def fire_gather(i, b):
            pltpu.async_copy(q_hbm.at[didx.at[i]], qbuf[b], gq[b])
            pltpu.async_copy(kt_hbm.at[sidx.at[i]], kbuf[b], gk_[b])

        def wait_gather(i, b):
            pltpu.make_async_copy(q_hbm.at[didx.at[i]], qbuf[b], gq[b]).wait()
            pltpu.make_async_copy(kt_hbm.at[sidx.at[i]], kbuf[b], gk_[b]).wait()

        def fire_write(i, b):
            base = base0 + i * CHG
            pltpu.async_copy(qbuf[b], qd_out.at[pl.ds(base, CHG)], wq[b])
            pltpu.async_copy(kbuf[b], g_out.at[pl.ds(base, CHG)], wk[b])

        def wait_write(i, b):
            base = base0 + i * CHG
            pltpu.make_async_copy(qbuf[b], qd_out.at[pl.ds(base, CHG)], wq[b]).wait()
            pltpu.make_async_copy(kbuf[b], g_out.at[pl.ds(base, CHG)], wk[b]).wait()

        fire_gather(0, 0)
        fire_gather(1, 1)

        def pair(j, carry):
            i0 = 2 * j
            for b in range(2):
                i = i0 + b
                wait_gather(i, b)
                fire_write(i, b)
            for b in range(2):
                i = i0 + b
                wait_write(i, b)

                @pl.when(i + 2 < NCHG)
                def _():
                    fire_gather(i + 2, b)
            return carry

        lax.fori_loop(0, NCHG // 2, pair, 0)
        last = NCHG - 1
        wait_gather(last, 0)
        fire_write(last, 0)
        wait_write(last, 0)

    return gk(q, kt, src3, dst3)


def _cut_body(r_ref, cut_ref):
    r = r_ref[...]
    cut_ref[...] = (0.5 * (jnp.cos(r * (math.pi / CUT)) + 1.0)
                    * (r < CUT).astype(jnp.float32))


def _cut_stage(rmat):
    return pl.pallas_call(
        _cut_body,
        out_shape=jax.ShapeDtypeStruct((E // D, D), jnp.float32),
    )(rmat)


BE = 1000


def _edge_body(qd_ref, g_ref, f_ref, cut_ref, d0_ref, d1_ref, d2_ref,
               wdk_ref, bdk_ref, wdv_ref, bdv_ref, m_ref,
               xe_ref, vm0_ref, vm1_ref, vm2_ref):
    dot = lambda a, w: jnp.dot(a, w, preferred_element_type=jnp.float32)
    f = f_ref[...]
    dk = _silu(dot(f, wdk_ref[...]) + bdk_ref[...])
    dvp = _silu(dot(f, wdv_ref[...]) + bdv_ref[...])
    attn_pre = dot(qd_ref[...] * g_ref[:, :D] * dk, m_ref[...])
    a = _silu(attn_pre) * cut_ref[...]
    vjx = g_ref[:, D:2 * D] * dvp[:, :D]
    v1e = g_ref[:, 2 * D:3 * D] * dvp[:, D:2 * D]
    v2e = g_ref[:, 3 * D:4 * D] * dvp[:, 2 * D:3 * D]
    xe_ref[...] = vjx * a
    for c, (dref, oref) in enumerate(((d0_ref, vm0_ref), (d1_ref, vm1_ref),
                                      (d2_ref, vm2_ref))):
        oref[...] = g_ref[:, 4 * D + c * D:4 * D + (c + 1) * D] * v1e + v2e * dref[...]


def _edge_stage(qd, g, f_ij, cut2, d0, d1, d2, Wdk, bdk2, Wdv_p, bdv_p2, m):
    def full(shape):
        return pl.BlockSpec(shape, lambda i: tuple(0 for _ in shape))
    row = lambda w: pl.BlockSpec((BE, w), lambda i: (i, 0))
    return pl.pallas_call(
        _edge_body,
        grid=(E // BE,),
        in_specs=[
            row(D), row(7 * D), row(R), row(1), row(1), row(1), row(1),
            full((R, D)), full((1, D)),
            full((R, 3 * D)), full((1, 3 * D)),
            full((D, D)),
        ],
        out_specs=[row(D), row(D), row(D), row(D)],
        out_shape=[jax.ShapeDtypeStruct((E, D), jnp.float32)] * 4,
    )(qd, g, f_ij, cut2, d0, d1, d2, Wdk, bdk2, Wdv_p, bdv_p2, m)


EPT = E // NS
CHS = 80
NCHS = EPT // CHS
NPAD = 10240
NPT = NPAD // NS


def _sc_scatter(m0, m1, m2, m3, dst3, zrows):
    mesh = plsc.VectorSubcoreMesh(core_axis_name="c", subcore_axis_name="s")

    @functools.partial(
        pl.kernel,
        mesh=mesh,
        out_type=[jax.ShapeDtypeStruct((NPAD, D), jnp.float32)] * 4,
        scratch_types=[
            pltpu.VMEM((NCHS, CHS), jnp.int32),
            [pltpu.VMEM((CHS, D), jnp.float32)] * 2,
            pltpu.VMEM_SHARED((NPAD, D), jnp.float32),
            [pltpu.SemaphoreType.DMA] * 2,
        ],
    )
    def sk(m0_hbm, m1_hbm, m2_hbm, m3_hbm, dst_hbm, z_hbm,
           o0, o1, o2, o3, idx, rows, acc, sems):
        cid = lax.axis_index("c")
        sid = lax.axis_index("s")
        row0 = sid * NPT
        pltpu.sync_copy(dst_hbm.at[sid], idx)
        planes = ((m0_hbm, o0), (m1_hbm, o1), (m2_hbm, o2), (m3_hbm, o3))
        for plane, (m_hbm, o_hbm) in enumerate(planes):
            @pl.when(cid == (plane // 2))
            def _(m_hbm=m_hbm, o_hbm=o_hbm):
                pltpu.sync_copy(z_hbm.at[pl.ds(row0, NPT)],
                                acc.at[pl.ds(row0, NPT)])

                def fire_read(i, b):
                    base = sid * EPT + i * CHS
                    pltpu.async_copy(m_hbm.at[pl.ds(base, CHS)],
                                     rows[b], sems[b])

                def wait_read(i, b):
                    base = sid * EPT + i * CHS
                    pltpu.make_async_copy(m_hbm.at[pl.ds(base, CHS)],
                                          rows[b], sems[b]).wait()

                plsc.subcore_barrier()
                fire_read(0, 0)
                fire_read(1, 1)

                def pair(j, carry):
                    i0 = 2 * j
                    for b in range(2):
                        i = i0 + b
                        wait_read(i, b)
                        pltpu.sync_copy(rows[b], acc.at[idx.at[i]], add=True)

                        @pl.when(i + 2 < NCHS)
                        def _():
                            fire_read(i + 2, b)
                    return carry

                lax.fori_loop(0, NCHS // 2, pair, 0)
                if NCHS % 2:
                    last = NCHS - 1
                    wait_read(last, 0)
                    pltpu.sync_copy(rows[0], acc.at[idx.at[last]], add=True)
                plsc.subcore_barrier()
                pltpu.sync_copy(acc.at[pl.ds(row0, NPT)],
                                o_hbm.at[pl.ds(row0, NPT)])
                plsc.subcore_barrier()

    return sk(m0, m1, m2, m3, dst3, zrows)


BN3 = 2000


def _final_body(xa_ref, a0_ref, a1_ref, a2_ref, vd_ref, v3_ref,
                wo_ref, bo_ref, dx_ref, dvec_ref):
    o = jnp.dot(xa_ref[...], wo_ref[...], preferred_element_type=jnp.float32) + bo_ref[...]
    o1 = o[:, :D]
    o2 = o[:, D:2 * D]
    o3 = o[:, 2 * D:3 * D]
    dx_ref[...] = vd_ref[...] * o2 + o3
    for c, aref in enumerate((a0_ref, a1_ref, a2_ref)):
        dvec_ref[:, c, :] = v3_ref[:, c, :] * o1 + aref[...]


def _final_stage(xagg, a0, a1, a2, vd, v3, Wo, bo2):
    def full(shape):
        return pl.BlockSpec(shape, lambda i: tuple(0 for _ in shape))
    row = lambda w: pl.BlockSpec((BN3, w), lambda i: (i, 0))
    return pl.pallas_call(
        _final_body,
        grid=(N // BN3,),
        in_specs=[
            row(D), row(D), row(D), row(D), row(D),
            pl.BlockSpec((BN3, 3, D), lambda i: (i, 0, 0)),
            full((D, 3 * D)), full((1, 3 * D)),
        ],
        out_specs=[
            row(D),
            pl.BlockSpec((BN3, 3, D), lambda i: (i, 0, 0)),
        ],
        out_shape=[
            jax.ShapeDtypeStruct((N, D), jnp.float32),
            jax.ShapeDtypeStruct((N, 3, D), jnp.float32),
        ],
    )(xagg, a0, a1, a2, vd, v3, Wo, bo2)


def kernel(x, vec, edge_index, r_ij, f_ij, d_ij, g_ln, b_ln, Wq, bq, Wk, bk,
           Wv, bv, Wvec, Wdk, bdk, Wdv, bdv, Wo, bo):
    f32 = jnp.float32
    Wv_p = Wv.reshape(D, H, 3, DH).transpose(0, 2, 1, 3).reshape(D, 3 * D)
    bv_p = bv.reshape(H, 3, DH).transpose(1, 0, 2).reshape(3 * D)
    Wdv_p = Wdv.reshape(R, H, 3, DH).transpose(0, 2, 1, 3).reshape(R, 3 * D)
    bdv_p = bdv.reshape(H, 3, DH).transpose(1, 0, 2).reshape(3 * D)
    src = edge_index[0]
    dst = edge_index[1]
    cut2 = _cut_stage(r_ij.reshape(E // D, D)).reshape(E, 1)
    d0 = d_ij[:, 0:1]
    d1 = d_ij[:, 1:2]
    d2 = d_ij[:, 2:3]
    m = jnp.kron(jnp.eye(H, dtype=f32), jnp.ones((DH, DH), f32))
    zrows = jnp.zeros((NPAD, D), f32)

    q, kt, vd, v3 = _node_stage(
        x, vec, g_ln.reshape(1, D), b_ln.reshape(1, D),
        Wq, bq.reshape(1, D), Wk, bk.reshape(1, D),
        Wv_p, bv_p.reshape(1, 3 * D), Wvec)
    qd, g = _sc_gather(q, kt, src.reshape(NW, NCHG, CHG),
                       dst.reshape(NW, NCHG, CHG))
    xe, vm0, vm1, vm2 = _edge_stage(
        qd, g, f_ij, cut2, d0, d1, d2,
        Wdk, bdk.reshape(1, D), Wdv_p, bdv_p.reshape(1, 3 * D), m)
    xagg, a0, a1, a2 = _sc_scatter(xe, vm0, vm1, vm2,
                                   dst.reshape(NS, NCHS, CHS), zrows)
    dx, dvec = _final_stage(xagg[:N], a0[:N], a1[:N], a2[:N], vd, v3,
                            Wo, bo.reshape(1, 3 * D))
    return dx, dvec


# speedup vs baseline: 37.1285x; 1.1717x over previous
import functools
import math

import jax
import jax.numpy as jnp
from jax import lax
from jax.experimental import pallas as pl
from jax.experimental.pallas import tpu as pltpu
from jax.experimental.pallas import tpu_sc as plsc

N = 10000
D = 128
H = 8
DH = 16
E = 160000
R = 64
CUT = 5.0

NC = 2
NS = 16
NW = NC * NS


BN1 = 1000


def _silu(a):
    return a / (1.0 + jnp.exp(-a))


def _node_body(x_ref, vec_ref, gln_ref, bln_ref, wq_ref, bq_ref, wvec_ref,
               q_ref, kt_ref, vd_ref, v3_ref):
    x = x_ref[...]
    mu = jnp.mean(x, axis=-1, keepdims=True)
    xc = x - mu
    var = jnp.mean(xc * xc, axis=-1, keepdims=True)
    xn = xc * lax.rsqrt(var + 1e-5) * gln_ref[...] + bln_ref[...]
    dot = lambda a, w: jnp.dot(a, w, preferred_element_type=jnp.float32)
    q_ref[...] = dot(xn, wq_ref[...]) + bq_ref[...]
    kt_ref[:, 0:D] = xn
    vd = jnp.zeros((x.shape[0], D), jnp.float32)
    for c in range(3):
        vc = vec_ref[:, c, :]
        vp = dot(vc, wvec_ref[...])
        vd = vd + vp[:, :D] * vp[:, D:2 * D]
        v3_ref[:, c, :] = vp[:, 2 * D:3 * D]
        kt_ref[:, D + c * D:D + (c + 1) * D] = vc
    vd_ref[...] = vd


def _node_stage(x, vec, g_ln2, b_ln2, Wq, bq2, Wvec):
    def full(shape):
        return pl.BlockSpec(shape, lambda i: tuple(0 for _ in shape))
    return pl.pallas_call(
        _node_body,
        grid=(N // BN1,),
        in_specs=[
            pl.BlockSpec((BN1, D), lambda i: (i, 0)),
            pl.BlockSpec((BN1, 3, D), lambda i: (i, 0, 0)),
            full((1, D)), full((1, D)),
            full((D, D)), full((1, D)),
            full((D, 3 * D)),
        ],
        out_specs=[
            pl.BlockSpec((BN1, D), lambda i: (i, 0)),
            pl.BlockSpec((BN1, 4 * D), lambda i: (i, 0)),
            pl.BlockSpec((BN1, D), lambda i: (i, 0)),
            pl.BlockSpec((BN1, 3, D), lambda i: (i, 0, 0)),
        ],
        out_shape=[
            jax.ShapeDtypeStruct((N, D), jnp.float32),
            jax.ShapeDtypeStruct((N, 4 * D), jnp.float32),
            jax.ShapeDtypeStruct((N, D), jnp.float32),
            jax.ShapeDtypeStruct((N, 3, D), jnp.float32),
        ],
    )(x, vec, g_ln2, b_ln2, Wq, bq2, Wvec)


EPW = E // NW
CHG = 40
NCHG = EPW // CHG


def _sc_gather(q, kt, src3, dst3):
    mesh = plsc.VectorSubcoreMesh(core_axis_name="c", subcore_axis_name="s")

    @functools.partial(
        pl.kernel,
        mesh=mesh,
        out_type=[jax.ShapeDtypeStruct((E, D), jnp.float32),
                  jax.ShapeDtypeStruct((E, 4 * D), jnp.float32)],
        scratch_types=[
            pltpu.VMEM((NCHG, CHG), jnp.int32),
            pltpu.VMEM((NCHG, CHG), jnp.int32),
            [pltpu.VMEM((CHG, D), jnp.float32)] * 3,
            [pltpu.VMEM((CHG, 4 * D), jnp.float32)] * 3,
            [pltpu.SemaphoreType.DMA] * 12,
        ],
    )
    def gk(q_hbm, kt_hbm, src_hbm, dst_hbm, qd_out, g_out,
           didx, sidx, qbuf, kbuf, sems):
        gq = sems[0:3]
        gk_ = sems[3:6]
        wq = sems[6:9]
        wk = sems[9:12]
        wid = lax.axis_index("s") * NC + lax.axis_index("c")
        base0 = wid * EPW
        pltpu.sync_copy(dst_hbm.at[wid], didx)
        pltpu.sync_copy(src_hbm.at[wid], sidx)

        def fire_gather(i, b):
            pltpu.async_copy(q_hbm.at[didx.at[i]], qbuf[b], gq[b])
            pltpu.async_copy(kt_hbm.at[sidx.at[i]], kbuf[b], gk_[b])

        def wait_gather(i, b):
            pltpu.make_async_copy(q_hbm.at[didx.at[i]], qbuf[b], gq[b]).wait()
            pltpu.make_async_copy(kt_hbm.at[sidx.at[i]], kbuf[b], gk_[b]).wait()

        def fire_write(i, b):
            base = base0 + i * CHG
            pltpu.async_copy(qbuf[b], qd_out.at[pl.ds(base, CHG)], wq[b])
            pltpu.async_copy(kbuf[b], g_out.at[pl.ds(base, CHG)], wk[b])

        def wait_write(i, b):
            base = base0 + i * CHG
            pltpu.make_async_copy(qbuf[b], qd_out.at[pl.ds(base, CHG)], wq[b]).wait()
            pltpu.make_async_copy(kbuf[b], g_out.at[pl.ds(base, CHG)], wk[b]).wait()

        fire_gather(0, 0)
        fire_gather(1, 1)
        fire_gather(2, 2)

        def triple(j, carry):
            i0 = 3 * j
            for b in range(3):
                i = i0 + b

                @pl.when(i < NCHG)
                def _(i=i, b=b):
                    wait_gather(i, b)
                    fire_write(i, b)
                    bp = (b + 2) % 3

                    @pl.when(jnp.logical_and(i >= 1, i + 2 < NCHG))
                    def _():
                        wait_write(i - 1, bp)
                        fire_gather(i + 2, bp)
            return carry

        lax.fori_loop(0, (NCHG + 2) // 3, triple, 0)
        for k in (NCHG - 3, NCHG - 2, NCHG - 1):
            wait_write(k, k % 3)

    return gk(q, kt, src3, dst3)


def _cut_body(r_ref, cut_ref):
    r = r_ref[...]
    cut_ref[...] = (0.5 * (jnp.cos(r * (math.pi / CUT)) + 1.0)
                    * (r < CUT).astype(jnp.float32))


def _cut_stage(rmat):
    return pl.pallas_call(
        _cut_body,
        out_shape=jax.ShapeDtypeStruct((E // D, D), jnp.float32),
    )(rmat)


BE = 1000


def _edge_body(qd_ref, g_ref, f_ref, cut_ref, d0_ref, d1_ref, d2_ref,
               wdk_ref, bdk_ref, wdv_ref, bdv_ref, wk_ref, bk_ref,
               wv_ref, bv_ref, m_ref,
               xe_ref, vm0_ref, vm1_ref, vm2_ref):
    dot = lambda a, w: jnp.dot(a, w, preferred_element_type=jnp.float32)
    f = f_ref[...]
    dk = _silu(dot(f, wdk_ref[...]) + bdk_ref[...])
    dvp = _silu(dot(f, wdv_ref[...]) + bdv_ref[...])
    xn_s = g_ref[:, :D]
    ks = dot(xn_s, wk_ref[...]) + bk_ref[...]
    vs = dot(xn_s, wv_ref[...]) + bv_ref[...]
    attn_pre = dot(qd_ref[...] * ks * dk, m_ref[...])
    a = _silu(attn_pre) * cut_ref[...]
    vjx = vs[:, :D] * dvp[:, :D]
    v1e = vs[:, D:2 * D] * dvp[:, D:2 * D]
    v2e = vs[:, 2 * D:3 * D] * dvp[:, 2 * D:3 * D]
    xe_ref[...] = vjx * a
    for c, (dref, oref) in enumerate(((d0_ref, vm0_ref), (d1_ref, vm1_ref),
                                      (d2_ref, vm2_ref))):
        oref[...] = g_ref[:, D + c * D:D + (c + 1) * D] * v1e + v2e * dref[...]


def _edge_stage(qd, g, f_ij, cut2, d0, d1, d2, Wdk, bdk2, Wdv_p, bdv_p2,
                Wk, bk2, Wv_p, bv_p2, m):
    def full(shape):
        return pl.BlockSpec(shape, lambda i: tuple(0 for _ in shape))
    row = lambda w: pl.BlockSpec((BE, w), lambda i: (i, 0))
    return pl.pallas_call(
        _edge_body,
        grid=(E // BE,),
        in_specs=[
            row(D), row(4 * D), row(R), row(1), row(1), row(1), row(1),
            full((R, D)), full((1, D)),
            full((R, 3 * D)), full((1, 3 * D)),
            full((D, D)), full((1, D)),
            full((D, 3 * D)), full((1, 3 * D)),
            full((D, D)),
        ],
        out_specs=[row(D), row(D), row(D), row(D)],
        out_shape=[jax.ShapeDtypeStruct((E, D), jnp.float32)] * 4,
    )(qd, g, f_ij, cut2, d0, d1, d2, Wdk, bdk2, Wdv_p, bdv_p2,
      Wk, bk2, Wv_p, bv_p2, m)


EPT = E // NS
CHS = 80
NCHS = EPT // CHS
NPAD = 10240
NPT = NPAD // NS


def _sc_scatter(m0, m1, m2, m3, dst3, zrows):
    mesh = plsc.VectorSubcoreMesh(core_axis_name="c", subcore_axis_name="s")

    @functools.partial(
        pl.kernel,
        mesh=mesh,
        out_type=[jax.ShapeDtypeStruct((NPAD, D), jnp.float32)] * 4,
        scratch_types=[
            pltpu.VMEM((NCHS, CHS), jnp.int32),
            [pltpu.VMEM((CHS, D), jnp.float32)] * 2,
            pltpu.VMEM_SHARED((NPAD, D), jnp.float32),
            [pltpu.SemaphoreType.DMA] * 2,
        ],
    )
    def sk(m0_hbm, m1_hbm, m2_hbm, m3_hbm, dst_hbm, z_hbm,
           o0, o1, o2, o3, idx, rows, acc, sems):
        cid = lax.axis_index("c")
        sid = lax.axis_index("s")
        row0 = sid * NPT
        pltpu.sync_copy(dst_hbm.at[sid], idx)
        planes = ((m0_hbm, o0), (m1_hbm, o1), (m2_hbm, o2), (m3_hbm, o3))
        for plane, (m_hbm, o_hbm) in enumerate(planes):
            @pl.when(cid == (plane // 2))
            def _(m_hbm=m_hbm, o_hbm=o_hbm):
                pltpu.sync_copy(z_hbm.at[pl.ds(row0, NPT)],
                                acc.at[pl.ds(row0, NPT)])

                def fire_read(i, b):
                    base = sid * EPT + i * CHS
                    pltpu.async_copy(m_hbm.at[pl.ds(base, CHS)],
                                     rows[b], sems[b])

                def wait_read(i, b):
                    base = sid * EPT + i * CHS
                    pltpu.make_async_copy(m_hbm.at[pl.ds(base, CHS)],
                                          rows[b], sems[b]).wait()

                plsc.subcore_barrier()
                fire_read(0, 0)
                fire_read(1, 1)

                def pair(j, carry):
                    i0 = 2 * j
                    for b in range(2):
                        i = i0 + b
                        wait_read(i, b)
                        pltpu.sync_copy(rows[b], acc.at[idx.at[i]], add=True)

                        @pl.when(i + 2 < NCHS)
                        def _():
                            fire_read(i + 2, b)
                    return carry

                lax.fori_loop(0, NCHS // 2, pair, 0)
                if NCHS % 2:
                    last = NCHS - 1
                    wait_read(last, 0)
                    pltpu.sync_copy(rows[0], acc.at[idx.at[last]], add=True)
                plsc.subcore_barrier()
                pltpu.sync_copy(acc.at[pl.ds(row0, NPT)],
                                o_hbm.at[pl.ds(row0, NPT)])
                plsc.subcore_barrier()

    return sk(m0, m1, m2, m3, dst3, zrows)


BN3 = 2000


def _final_body(xa_ref, a0_ref, a1_ref, a2_ref, vd_ref, v3_ref,
                wo_ref, bo_ref, dx_ref, dvec_ref):
    o = jnp.dot(xa_ref[...], wo_ref[...], preferred_element_type=jnp.float32) + bo_ref[...]
    o1 = o[:, :D]
    o2 = o[:, D:2 * D]
    o3 = o[:, 2 * D:3 * D]
    dx_ref[...] = vd_ref[...] * o2 + o3
    for c, aref in enumerate((a0_ref, a1_ref, a2_ref)):
        dvec_ref[:, c, :] = v3_ref[:, c, :] * o1 + aref[...]


def _final_stage(xagg, a0, a1, a2, vd, v3, Wo, bo2):
    def full(shape):
        return pl.BlockSpec(shape, lambda i: tuple(0 for _ in shape))
    row = lambda w: pl.BlockSpec((BN3, w), lambda i: (i, 0))
    return pl.pallas_call(
        _final_body,
        grid=(N // BN3,),
        in_specs=[
            row(D), row(D), row(D), row(D), row(D),
            pl.BlockSpec((BN3, 3, D), lambda i: (i, 0, 0)),
            full((D, 3 * D)), full((1, 3 * D)),
        ],
        out_specs=[
            row(D),
            pl.BlockSpec((BN3, 3, D), lambda i: (i, 0, 0)),
        ],
        out_shape=[
            jax.ShapeDtypeStruct((N, D), jnp.float32),
            jax.ShapeDtypeStruct((N, 3, D), jnp.float32),
        ],
    )(xagg, a0, a1, a2, vd, v3, Wo, bo2)


def kernel(x, vec, edge_index, r_ij, f_ij, d_ij, g_ln, b_ln, Wq, bq, Wk, bk,
           Wv, bv, Wvec, Wdk, bdk, Wdv, bdv, Wo, bo):
    f32 = jnp.float32
    Wv_p = Wv.reshape(D, H, 3, DH).transpose(0, 2, 1, 3).reshape(D, 3 * D)
    bv_p = bv.reshape(H, 3, DH).transpose(1, 0, 2).reshape(3 * D)
    Wdv_p = Wdv.reshape(R, H, 3, DH).transpose(0, 2, 1, 3).reshape(R, 3 * D)
    bdv_p = bdv.reshape(H, 3, DH).transpose(1, 0, 2).reshape(3 * D)
    src = edge_index[0]
    dst = edge_index[1]
    cut2 = _cut_stage(r_ij.reshape(E // D, D)).reshape(E, 1)
    d0 = d_ij[:, 0:1]
    d1 = d_ij[:, 1:2]
    d2 = d_ij[:, 2:3]
    m = jnp.kron(jnp.eye(H, dtype=f32), jnp.ones((DH, DH), f32))
    zrows = jnp.zeros((NPAD, D), f32)

    q, kt, vd, v3 = _node_stage(
        x, vec, g_ln.reshape(1, D), b_ln.reshape(1, D),
        Wq, bq.reshape(1, D), Wvec)
    qd, g = _sc_gather(q, kt, src.reshape(NW, NCHG, CHG),
                       dst.reshape(NW, NCHG, CHG))
    xe, vm0, vm1, vm2 = _edge_stage(
        qd, g, f_ij, cut2, d0, d1, d2,
        Wdk, bdk.reshape(1, D), Wdv_p, bdv_p.reshape(1, 3 * D),
        Wk, bk.reshape(1, D), Wv_p, bv_p.reshape(1, 3 * D), m)
    xagg, a0, a1, a2 = _sc_scatter(xe, vm0, vm1, vm2,
                                   dst.reshape(NS, NCHS, CHS), zrows)
    dx, dvec = _final_stage(xagg[:N], a0[:N], a1[:N], a2[:N], vd, v3,
                            Wo, bo.reshape(1, 3 * D))
    return dx, dvec


# speedup vs baseline: 37.4071x; 1.0075x over previous
import functools
import math

import jax
import jax.numpy as jnp
from jax import lax
from jax.experimental import pallas as pl
from jax.experimental.pallas import tpu as pltpu
from jax.experimental.pallas import tpu_sc as plsc

N = 10000
D = 128
H = 8
DH = 16
E = 160000
R = 64
CUT = 5.0

NC = 2
NS = 16
NW = NC * NS


BN1 = 1000


def _silu(a):
    return a / (1.0 + jnp.exp(-a))


def _node_body(x_ref, vec_ref, gln_ref, bln_ref, wq_ref, bq_ref, wvec_ref,
               q_ref, kt_ref, vd_ref, v3_ref):
    x = x_ref[...]
    mu = jnp.mean(x, axis=-1, keepdims=True)
    xc = x - mu
    var = jnp.mean(xc * xc, axis=-1, keepdims=True)
    xn = xc * lax.rsqrt(var + 1e-5) * gln_ref[...] + bln_ref[...]
    dot = lambda a, w: jnp.dot(a, w, preferred_element_type=jnp.float32)
    q_ref[...] = dot(xn, wq_ref[...]) + bq_ref[...]
    kt_ref[:, 0:D] = xn
    vd = jnp.zeros((x.shape[0], D), jnp.float32)
    for c in range(3):
        vc = vec_ref[:, c, :]
        vp = dot(vc, wvec_ref[...])
        vd = vd + vp[:, :D] * vp[:, D:2 * D]
        v3_ref[:, c, :] = vp[:, 2 * D:3 * D]
        kt_ref[:, D + c * D:D + (c + 1) * D] = vc
    vd_ref[...] = vd


def _node_stage(x, vec, g_ln2, b_ln2, Wq, bq2, Wvec):
    def full(shape):
        return pl.BlockSpec(shape, lambda i: tuple(0 for _ in shape))
    return pl.pallas_call(
        _node_body,
        grid=(N // BN1,),
        in_specs=[
            pl.BlockSpec((BN1, D), lambda i: (i, 0)),
            pl.BlockSpec((BN1, 3, D), lambda i: (i, 0, 0)),
            full((1, D)), full((1, D)),
            full((D, D)), full((1, D)),
            full((D, 3 * D)),
        ],
        out_specs=[
            pl.BlockSpec((BN1, D), lambda i: (i, 0)),
            pl.BlockSpec((BN1, 4 * D), lambda i: (i, 0)),
            pl.BlockSpec((BN1, D), lambda i: (i, 0)),
            pl.BlockSpec((BN1, 3, D), lambda i: (i, 0, 0)),
        ],
        out_shape=[
            jax.ShapeDtypeStruct((N, D), jnp.float32),
            jax.ShapeDtypeStruct((N, 4 * D), jnp.float32),
            jax.ShapeDtypeStruct((N, D), jnp.float32),
            jax.ShapeDtypeStruct((N, 3, D), jnp.float32),
        ],
    )(x, vec, g_ln2, b_ln2, Wq, bq2, Wvec)


EPW = E // NW
CHG = 80
NCHF = EPW // CHG
RESG = EPW - NCHF * CHG


def _sc_gather(q, kt, src, dst):
    mesh = plsc.VectorSubcoreMesh(core_axis_name="c", subcore_axis_name="s")

    @functools.partial(
        pl.kernel,
        mesh=mesh,
        out_type=[jax.ShapeDtypeStruct((E, D), jnp.float32),
                  jax.ShapeDtypeStruct((E, 4 * D), jnp.float32)],
        scratch_types=[
            pltpu.VMEM((EPW,), jnp.int32),
            pltpu.VMEM((EPW,), jnp.int32),
            [pltpu.VMEM((CHG, D), jnp.float32)] * 2,
            [pltpu.VMEM((CHG, 4 * D), jnp.float32)] * 2,
            [pltpu.SemaphoreType.DMA] * 8,
        ],
    )
    def gk(q_hbm, kt_hbm, src_hbm, dst_hbm, qd_out, g_out,
           didx, sidx, qbuf, kbuf, sems):
        gq = sems[0:2]
        gk_ = sems[2:4]
        wq = sems[4:6]
        wk = sems[6:8]
        wid = lax.axis_index("s") * NC + lax.axis_index("c")
        base0 = wid * EPW
        pltpu.sync_copy(dst_hbm.at[pl.ds(base0, EPW)], didx)
        pltpu.sync_copy(src_hbm.at[pl.ds(base0, EPW)], sidx)

        def fire_gather(i, b, n=CHG):
            pltpu.async_copy(q_hbm.at[didx.at[pl.ds(i * CHG, n)]],
                             qbuf[b].at[pl.ds(0, n)], gq[b])
            pltpu.async_copy(kt_hbm.at[sidx.at[pl.ds(i * CHG, n)]],
                             kbuf[b].at[pl.ds(0, n)], gk_[b])

        def wait_gather(i, b, n=CHG):
            pltpu.make_async_copy(q_hbm.at[didx.at[pl.ds(i * CHG, n)]],
                                  qbuf[b].at[pl.ds(0, n)], gq[b]).wait()
            pltpu.make_async_copy(kt_hbm.at[sidx.at[pl.ds(i * CHG, n)]],
                                  kbuf[b].at[pl.ds(0, n)], gk_[b]).wait()

        def fire_write(i, b, n=CHG):
            base = base0 + i * CHG
            pltpu.async_copy(qbuf[b].at[pl.ds(0, n)],
                             qd_out.at[pl.ds(base, n)], wq[b])
            pltpu.async_copy(kbuf[b].at[pl.ds(0, n)],
                             g_out.at[pl.ds(base, n)], wk[b])

        def wait_write(i, b, n=CHG):
            base = base0 + i * CHG
            pltpu.make_async_copy(qbuf[b].at[pl.ds(0, n)],
                                  qd_out.at[pl.ds(base, n)], wq[b]).wait()
            pltpu.make_async_copy(kbuf[b].at[pl.ds(0, n)],
                                  g_out.at[pl.ds(base, n)], wk[b]).wait()

        fire_gather(0, 0)

        def pair(j, carry):
            i0 = 2 * j
            for b in range(2):
                i = i0 + b
                wait_gather(i, b)
                fire_write(i, b)
                bn = 1 - b

                @pl.when(i == 0)
                def _():
                    fire_gather(1, 1)

                @pl.when(jnp.logical_and(i >= 1, i + 1 < NCHF))
                def _(i=i, bn=bn):
                    wait_write(i - 1, bn)
                    fire_gather(i + 1, bn)
            return carry

        lax.fori_loop(0, NCHF // 2, pair, 0)
        wait_write(NCHF - 2, 0)
        wait_write(NCHF - 1, 1)
        if RESG:
            fire_gather(NCHF, 0, RESG)
            wait_gather(NCHF, 0, RESG)
            fire_write(NCHF, 0, RESG)
            wait_write(NCHF, 0, RESG)

    return gk(q, kt, src, dst)


def _cut_body(r_ref, cut_ref):
    r = r_ref[...]
    cut_ref[...] = (0.5 * (jnp.cos(r * (math.pi / CUT)) + 1.0)
                    * (r < CUT).astype(jnp.float32))


def _cut_stage(rmat):
    return pl.pallas_call(
        _cut_body,
        out_shape=jax.ShapeDtypeStruct((E // D, D), jnp.float32),
    )(rmat)


BE = 1000


def _edge_body(qd_ref, g_ref, f_ref, cut_ref, d0_ref, d1_ref, d2_ref,
               wdk_ref, bdk_ref, wdv_ref, bdv_ref, wk_ref, bk_ref,
               wv_ref, bv_ref, m_ref,
               xe_ref, vm0_ref, vm1_ref, vm2_ref):
    dot = lambda a, w: jnp.dot(a, w, preferred_element_type=jnp.float32)
    f = f_ref[...]
    dk = _silu(dot(f, wdk_ref[...]) + bdk_ref[...])
    dvp = _silu(dot(f, wdv_ref[...]) + bdv_ref[...])
    xn_s = g_ref[:, :D]
    ks = dot(xn_s, wk_ref[...]) + bk_ref[...]
    vs = dot(xn_s, wv_ref[...]) + bv_ref[...]
    attn_pre = dot(qd_ref[...] * ks * dk, m_ref[...])
    a = _silu(attn_pre) * cut_ref[...]
    vjx = vs[:, :D] * dvp[:, :D]
    v1e = vs[:, D:2 * D] * dvp[:, D:2 * D]
    v2e = vs[:, 2 * D:3 * D] * dvp[:, 2 * D:3 * D]
    xe_ref[...] = vjx * a
    for c, (dref, oref) in enumerate(((d0_ref, vm0_ref), (d1_ref, vm1_ref),
                                      (d2_ref, vm2_ref))):
        oref[...] = g_ref[:, D + c * D:D + (c + 1) * D] * v1e + v2e * dref[...]


def _edge_stage(qd, g, f_ij, cut2, d0, d1, d2, Wdk, bdk2, Wdv_p, bdv_p2,
                Wk, bk2, Wv_p, bv_p2, m):
    def full(shape):
        return pl.BlockSpec(shape, lambda i: tuple(0 for _ in shape))
    row = lambda w: pl.BlockSpec((BE, w), lambda i: (i, 0))
    return pl.pallas_call(
        _edge_body,
        grid=(E // BE,),
        in_specs=[
            row(D), row(4 * D), row(R), row(1), row(1), row(1), row(1),
            full((R, D)), full((1, D)),
            full((R, 3 * D)), full((1, 3 * D)),
            full((D, D)), full((1, D)),
            full((D, 3 * D)), full((1, 3 * D)),
            full((D, D)),
        ],
        out_specs=[row(D), row(D), row(D), row(D)],
        out_shape=[jax.ShapeDtypeStruct((E, D), jnp.float32)] * 4,
    )(qd, g, f_ij, cut2, d0, d1, d2, Wdk, bdk2, Wdv_p, bdv_p2,
      Wk, bk2, Wv_p, bv_p2, m)


EPT = E // NS
CHS = 80
NCHS = EPT // CHS
NPAD = 10240
NPT = NPAD // NS


def _sc_scatter(m0, m1, m2, m3, dst3, zrows):
    mesh = plsc.VectorSubcoreMesh(core_axis_name="c", subcore_axis_name="s")

    @functools.partial(
        pl.kernel,
        mesh=mesh,
        out_type=[jax.ShapeDtypeStruct((NPAD, D), jnp.float32)] * 4,
        scratch_types=[
            pltpu.VMEM((NCHS, CHS), jnp.int32),
            [pltpu.VMEM((CHS, D), jnp.float32)] * 2,
            pltpu.VMEM_SHARED((NPAD, D), jnp.float32),
            [pltpu.SemaphoreType.DMA] * 2,
        ],
    )
    def sk(m0_hbm, m1_hbm, m2_hbm, m3_hbm, dst_hbm, z_hbm,
           o0, o1, o2, o3, idx, rows, acc, sems):
        cid = lax.axis_index("c")
        sid = lax.axis_index("s")
        row0 = sid * NPT
        pltpu.sync_copy(dst_hbm.at[sid], idx)
        planes = ((m0_hbm, o0), (m1_hbm, o1), (m2_hbm, o2), (m3_hbm, o3))
        for plane, (m_hbm, o_hbm) in enumerate(planes):
            @pl.when(cid == (plane // 2))
            def _(m_hbm=m_hbm, o_hbm=o_hbm):
                pltpu.sync_copy(z_hbm.at[pl.ds(row0, NPT)],
                                acc.at[pl.ds(row0, NPT)])

                def fire_read(i, b):
                    base = sid * EPT + i * CHS
                    pltpu.async_copy(m_hbm.at[pl.ds(base, CHS)],
                                     rows[b], sems[b])

                def wait_read(i, b):
                    base = sid * EPT + i * CHS
                    pltpu.make_async_copy(m_hbm.at[pl.ds(base, CHS)],
                                          rows[b], sems[b]).wait()

                plsc.subcore_barrier()
                fire_read(0, 0)
                fire_read(1, 1)

                def pair(j, carry):
                    i0 = 2 * j
                    for b in range(2):
                        i = i0 + b
                        wait_read(i, b)
                        pltpu.sync_copy(rows[b], acc.at[idx.at[i]], add=True)

                        @pl.when(i + 2 < NCHS)
                        def _():
                            fire_read(i + 2, b)
                    return carry

                lax.fori_loop(0, NCHS // 2, pair, 0)
                if NCHS % 2:
                    last = NCHS - 1
                    wait_read(last, 0)
                    pltpu.sync_copy(rows[0], acc.at[idx.at[last]], add=True)
                plsc.subcore_barrier()
                pltpu.sync_copy(acc.at[pl.ds(row0, NPT)],
                                o_hbm.at[pl.ds(row0, NPT)])
                plsc.subcore_barrier()

    return sk(m0, m1, m2, m3, dst3, zrows)


BN3 = 2000


def _final_body(xa_ref, a0_ref, a1_ref, a2_ref, vd_ref, v3_ref,
                wo_ref, bo_ref, dx_ref, dvec_ref):
    o = jnp.dot(xa_ref[...], wo_ref[...], preferred_element_type=jnp.float32) + bo_ref[...]
    o1 = o[:, :D]
    o2 = o[:, D:2 * D]
    o3 = o[:, 2 * D:3 * D]
    dx_ref[...] = vd_ref[...] * o2 + o3
    for c, aref in enumerate((a0_ref, a1_ref, a2_ref)):
        dvec_ref[:, c, :] = v3_ref[:, c, :] * o1 + aref[...]


def _final_stage(xagg, a0, a1, a2, vd, v3, Wo, bo2):
    def full(shape):
        return pl.BlockSpec(shape, lambda i: tuple(0 for _ in shape))
    row = lambda w: pl.BlockSpec((BN3, w), lambda i: (i, 0))
    return pl.pallas_call(
        _final_body,
        grid=(N // BN3,),
        in_specs=[
            row(D), row(D), row(D), row(D), row(D),
            pl.BlockSpec((BN3, 3, D), lambda i: (i, 0, 0)),
            full((D, 3 * D)), full((1, 3 * D)),
        ],
        out_specs=[
            row(D),
            pl.BlockSpec((BN3, 3, D), lambda i: (i, 0, 0)),
        ],
        out_shape=[
            jax.ShapeDtypeStruct((N, D), jnp.float32),
            jax.ShapeDtypeStruct((N, 3, D), jnp.float32),
        ],
    )(xagg, a0, a1, a2, vd, v3, Wo, bo2)


def kernel(x, vec, edge_index, r_ij, f_ij, d_ij, g_ln, b_ln, Wq, bq, Wk, bk,
           Wv, bv, Wvec, Wdk, bdk, Wdv, bdv, Wo, bo):
    f32 = jnp.float32
    Wv_p = Wv.reshape(D, H, 3, DH).transpose(0, 2, 1, 3).reshape(D, 3 * D)
    bv_p = bv.reshape(H, 3, DH).transpose(1, 0, 2).reshape(3 * D)
    Wdv_p = Wdv.reshape(R, H, 3, DH).transpose(0, 2, 1, 3).reshape(R, 3 * D)
    bdv_p = bdv.reshape(H, 3, DH).transpose(1, 0, 2).reshape(3 * D)
    src = edge_index[0]
    dst = edge_index[1]
    cut2 = _cut_stage(r_ij.reshape(E // D, D)).reshape(E, 1)
    d0 = d_ij[:, 0:1]
    d1 = d_ij[:, 1:2]
    d2 = d_ij[:, 2:3]
    m = jnp.kron(jnp.eye(H, dtype=f32), jnp.ones((DH, DH), f32))
    zrows = jnp.zeros((NPAD, D), f32)

    q, kt, vd, v3 = _node_stage(
        x, vec, g_ln.reshape(1, D), b_ln.reshape(1, D),
        Wq, bq.reshape(1, D), Wvec)
    qd, g = _sc_gather(q, kt, src, dst)
    xe, vm0, vm1, vm2 = _edge_stage(
        qd, g, f_ij, cut2, d0, d1, d2,
        Wdk, bdk.reshape(1, D), Wdv_p, bdv_p.reshape(1, 3 * D),
        Wk, bk.reshape(1, D), Wv_p, bv_p.reshape(1, 3 * D), m)
    xagg, a0, a1, a2 = _sc_scatter(xe, vm0, vm1, vm2,
                                   dst.reshape(NS, NCHS, CHS), zrows)
    dx, dvec = _final_stage(xagg[:N], a0[:N], a1[:N], a2[:N], vd, v3,
                            Wo, bo.reshape(1, 3 * D))
    return dx, dvec


# speedup vs baseline: 39.4595x; 1.0549x over previous
import functools
import math

import jax
import jax.numpy as jnp
from jax import lax
from jax.experimental import pallas as pl
from jax.experimental.pallas import tpu as pltpu
from jax.experimental.pallas import tpu_sc as plsc

N = 10000
D = 128
H = 8
DH = 16
E = 160000
R = 64
CUT = 5.0

NC = 2
NS = 16
NW = NC * NS


BN1 = 1000


def _silu(a):
    return a / (1.0 + jnp.exp(-a))


def _pack_bf(x):
    u = jax.lax.bitcast_convert_type(x, jnp.uint32)
    r = (u + jnp.uint32(0x7FFF) + ((u >> 16) & jnp.uint32(1))) >> 16
    lo = r[:, :D // 2]
    hi = r[:, D // 2:]
    return jax.lax.bitcast_convert_type(lo | (hi << 16), jnp.int32)


def _unpack_bf(w):
    u = jax.lax.bitcast_convert_type(w, jnp.uint32)
    lo = jax.lax.bitcast_convert_type(u << 16, jnp.float32)
    hi = jax.lax.bitcast_convert_type(u & jnp.uint32(0xFFFF0000), jnp.float32)
    return jnp.concatenate([lo, hi], axis=1)


def _node_body(x_ref, vec_ref, gln_ref, bln_ref, wq_ref, bq_ref, wvec_ref,
               q_ref, kt_ref, vd_ref, v3_ref):
    x = x_ref[...]
    mu = jnp.mean(x, axis=-1, keepdims=True)
    xc = x - mu
    var = jnp.mean(xc * xc, axis=-1, keepdims=True)
    xn = xc * lax.rsqrt(var + 1e-5) * gln_ref[...] + bln_ref[...]
    dot = lambda a, w: jnp.dot(a, w, preferred_element_type=jnp.float32)
    HD = D // 2
    q_ref[...] = dot(xn, wq_ref[...]) + bq_ref[...]
    kt_ref[:, 0:HD] = _pack_bf(xn)
    vd = jnp.zeros((x.shape[0], D), jnp.float32)
    for c in range(3):
        vc = vec_ref[:, c, :]
        vp = dot(vc, wvec_ref[...])
        vd = vd + vp[:, :D] * vp[:, D:2 * D]
        v3_ref[:, c, :] = vp[:, 2 * D:3 * D]
        kt_ref[:, HD + c * HD:HD + (c + 1) * HD] = _pack_bf(vc)
    vd_ref[...] = vd


def _node_stage(x, vec, g_ln2, b_ln2, Wq, bq2, Wvec):
    def full(shape):
        return pl.BlockSpec(shape, lambda i: tuple(0 for _ in shape))
    return pl.pallas_call(
        _node_body,
        grid=(N // BN1,),
        in_specs=[
            pl.BlockSpec((BN1, D), lambda i: (i, 0)),
            pl.BlockSpec((BN1, 3, D), lambda i: (i, 0, 0)),
            full((1, D)), full((1, D)),
            full((D, D)), full((1, D)),
            full((D, 3 * D)),
        ],
        out_specs=[
            pl.BlockSpec((BN1, D), lambda i: (i, 0)),
            pl.BlockSpec((BN1, 2 * D), lambda i: (i, 0)),
            pl.BlockSpec((BN1, D), lambda i: (i, 0)),
            pl.BlockSpec((BN1, 3, D), lambda i: (i, 0, 0)),
        ],
        out_shape=[
            jax.ShapeDtypeStruct((N, D), jnp.float32),
            jax.ShapeDtypeStruct((N, 2 * D), jnp.int32),
            jax.ShapeDtypeStruct((N, D), jnp.float32),
            jax.ShapeDtypeStruct((N, 3, D), jnp.float32),
        ],
    )(x, vec, g_ln2, b_ln2, Wq, bq2, Wvec)


EPW = E // NW
CHG = 80
NCHF = EPW // CHG
RESG = EPW - NCHF * CHG


def _sc_gather(q, kt, src, dst):
    mesh = plsc.VectorSubcoreMesh(core_axis_name="c", subcore_axis_name="s")

    @functools.partial(
        pl.kernel,
        mesh=mesh,
        out_type=[jax.ShapeDtypeStruct((E, D), jnp.float32),
                  jax.ShapeDtypeStruct((E, 2 * D), jnp.int32)],
        scratch_types=[
            pltpu.VMEM((EPW,), jnp.int32),
            pltpu.VMEM((EPW,), jnp.int32),
            [pltpu.VMEM((CHG, D), jnp.float32)] * 2,
            [pltpu.VMEM((CHG, 2 * D), jnp.int32)] * 2,
            [pltpu.SemaphoreType.DMA] * 8,
        ],
    )
    def gk(q_hbm, kt_hbm, src_hbm, dst_hbm, qd_out, g_out,
           didx, sidx, qbuf, kbuf, sems):
        gq = sems[0:2]
        gk_ = sems[2:4]
        wq = sems[4:6]
        wk = sems[6:8]
        wid = lax.axis_index("s") * NC + lax.axis_index("c")
        base0 = wid * EPW
        pltpu.sync_copy(dst_hbm.at[pl.ds(base0, EPW)], didx)
        pltpu.sync_copy(src_hbm.at[pl.ds(base0, EPW)], sidx)

        def fire_gather(i, b, n=CHG):
            pltpu.async_copy(q_hbm.at[didx.at[pl.ds(i * CHG, n)]],
                             qbuf[b].at[pl.ds(0, n)], gq[b])
            pltpu.async_copy(kt_hbm.at[sidx.at[pl.ds(i * CHG, n)]],
                             kbuf[b].at[pl.ds(0, n)], gk_[b])

        def wait_gather(i, b, n=CHG):
            pltpu.make_async_copy(q_hbm.at[didx.at[pl.ds(i * CHG, n)]],
                                  qbuf[b].at[pl.ds(0, n)], gq[b]).wait()
            pltpu.make_async_copy(kt_hbm.at[sidx.at[pl.ds(i * CHG, n)]],
                                  kbuf[b].at[pl.ds(0, n)], gk_[b]).wait()

        def fire_write(i, b, n=CHG):
            base = base0 + i * CHG
            pltpu.async_copy(qbuf[b].at[pl.ds(0, n)],
                             qd_out.at[pl.ds(base, n)], wq[b])
            pltpu.async_copy(kbuf[b].at[pl.ds(0, n)],
                             g_out.at[pl.ds(base, n)], wk[b])

        def wait_write(i, b, n=CHG):
            base = base0 + i * CHG
            pltpu.make_async_copy(qbuf[b].at[pl.ds(0, n)],
                                  qd_out.at[pl.ds(base, n)], wq[b]).wait()
            pltpu.make_async_copy(kbuf[b].at[pl.ds(0, n)],
                                  g_out.at[pl.ds(base, n)], wk[b]).wait()

        fire_gather(0, 0)

        def pair(j, carry):
            i0 = 2 * j
            for b in range(2):
                i = i0 + b
                wait_gather(i, b)
                fire_write(i, b)
                bn = 1 - b

                @pl.when(i == 0)
                def _():
                    fire_gather(1, 1)

                @pl.when(jnp.logical_and(i >= 1, i + 1 < NCHF))
                def _(i=i, bn=bn):
                    wait_write(i - 1, bn)
                    fire_gather(i + 1, bn)
            return carry

        lax.fori_loop(0, NCHF // 2, pair, 0)
        wait_write(NCHF - 2, 0)
        wait_write(NCHF - 1, 1)
        if RESG:
            fire_gather(NCHF, 0, RESG)
            wait_gather(NCHF, 0, RESG)
            fire_write(NCHF, 0, RESG)
            wait_write(NCHF, 0, RESG)

    return gk(q, kt, src, dst)


def _cut_body(r_ref, cut_ref):
    r = r_ref[...]
    cut_ref[...] = (0.5 * (jnp.cos(r * (math.pi / CUT)) + 1.0)
                    * (r < CUT).astype(jnp.float32))


def _cut_stage(rmat):
    return pl.pallas_call(
        _cut_body,
        out_shape=jax.ShapeDtypeStruct((E // D, D), jnp.float32),
    )(rmat)


BE = 1000


def _edge_body(qd_ref, g_ref, f_ref, cut_ref, d0_ref, d1_ref, d2_ref,
               wdk_ref, bdk_ref, wdv_ref, bdv_ref, wk_ref, bk_ref,
               wv_ref, bv_ref, m_ref,
               xe_ref, vm0_ref, vm1_ref, vm2_ref):
    dot = lambda a, w: jnp.dot(a, w, preferred_element_type=jnp.float32)
    f = f_ref[...]
    dk = _silu(dot(f, wdk_ref[...]) + bdk_ref[...])
    dvp = _silu(dot(f, wdv_ref[...]) + bdv_ref[...])
    HD = D // 2
    xn_s = _unpack_bf(g_ref[:, :HD])
    ks = dot(xn_s, wk_ref[...]) + bk_ref[...]
    vs = dot(xn_s, wv_ref[...]) + bv_ref[...]
    attn_pre = dot(qd_ref[...] * ks * dk, m_ref[...])
    a = _silu(attn_pre) * cut_ref[...]
    vjx = vs[:, :D] * dvp[:, :D]
    v1e = vs[:, D:2 * D] * dvp[:, D:2 * D]
    v2e = vs[:, 2 * D:3 * D] * dvp[:, 2 * D:3 * D]
    xe_ref[...] = vjx * a
    for c, (dref, oref) in enumerate(((d0_ref, vm0_ref), (d1_ref, vm1_ref),
                                      (d2_ref, vm2_ref))):
        vecc = _unpack_bf(g_ref[:, HD + c * HD:HD + (c + 1) * HD])
        oref[...] = vecc * v1e + v2e * dref[...]


def _edge_stage(qd, g, f_ij, cut2, d0, d1, d2, Wdk, bdk2, Wdv_p, bdv_p2,
                Wk, bk2, Wv_p, bv_p2, m):
    def full(shape):
        return pl.BlockSpec(shape, lambda i: tuple(0 for _ in shape))
    row = lambda w: pl.BlockSpec((BE, w), lambda i: (i, 0))
    return pl.pallas_call(
        _edge_body,
        grid=(E // BE,),
        in_specs=[
            row(D), row(2 * D), row(R), row(1), row(1), row(1), row(1),
            full((R, D)), full((1, D)),
            full((R, 3 * D)), full((1, 3 * D)),
            full((D, D)), full((1, D)),
            full((D, 3 * D)), full((1, 3 * D)),
            full((D, D)),
        ],
        out_specs=[row(D), row(D), row(D), row(D)],
        out_shape=[jax.ShapeDtypeStruct((E, D), jnp.float32)] * 4,
    )(qd, g, f_ij, cut2, d0, d1, d2, Wdk, bdk2, Wdv_p, bdv_p2,
      Wk, bk2, Wv_p, bv_p2, m)


EPT = E // NS
CHS = 80
NCHS = EPT // CHS
NPAD = 10240
NPT = NPAD // NS


def _sc_scatter(m0, m1, m2, m3, dst3, zrows):
    mesh = plsc.VectorSubcoreMesh(core_axis_name="c", subcore_axis_name="s")

    @functools.partial(
        pl.kernel,
        mesh=mesh,
        out_type=[jax.ShapeDtypeStruct((NPAD, D), jnp.float32)] * 4,
        scratch_types=[
            pltpu.VMEM((NCHS, CHS), jnp.int32),
            [pltpu.VMEM((CHS, D), jnp.float32)] * 2,
            pltpu.VMEM_SHARED((NPAD, D), jnp.float32),
            [pltpu.SemaphoreType.DMA] * 2,
        ],
    )
    def sk(m0_hbm, m1_hbm, m2_hbm, m3_hbm, dst_hbm, z_hbm,
           o0, o1, o2, o3, idx, rows, acc, sems):
        cid = lax.axis_index("c")
        sid = lax.axis_index("s")
        row0 = sid * NPT
        pltpu.sync_copy(dst_hbm.at[sid], idx)
        planes = ((m0_hbm, o0), (m1_hbm, o1), (m2_hbm, o2), (m3_hbm, o3))
        for plane, (m_hbm, o_hbm) in enumerate(planes):
            @pl.when(cid == (plane // 2))
            def _(m_hbm=m_hbm, o_hbm=o_hbm):
                pltpu.sync_copy(z_hbm.at[pl.ds(row0, NPT)],
                                acc.at[pl.ds(row0, NPT)])

                def fire_read(i, b):
                    base = sid * EPT + i * CHS
                    pltpu.async_copy(m_hbm.at[pl.ds(base, CHS)],
                                     rows[b], sems[b])

                def wait_read(i, b):
                    base = sid * EPT + i * CHS
                    pltpu.make_async_copy(m_hbm.at[pl.ds(base, CHS)],
                                          rows[b], sems[b]).wait()

                plsc.subcore_barrier()
                fire_read(0, 0)
                fire_read(1, 1)

                def pair(j, carry):
                    i0 = 2 * j
                    for b in range(2):
                        i = i0 + b
                        wait_read(i, b)
                        pltpu.sync_copy(rows[b], acc.at[idx.at[i]], add=True)

                        @pl.when(i + 2 < NCHS)
                        def _():
                            fire_read(i + 2, b)
                    return carry

                lax.fori_loop(0, NCHS // 2, pair, 0)
                if NCHS % 2:
                    last = NCHS - 1
                    wait_read(last, 0)
                    pltpu.sync_copy(rows[0], acc.at[idx.at[last]], add=True)
                plsc.subcore_barrier()
                pltpu.sync_copy(acc.at[pl.ds(row0, NPT)],
                                o_hbm.at[pl.ds(row0, NPT)])
                plsc.subcore_barrier()

    return sk(m0, m1, m2, m3, dst3, zrows)


BN3 = 2000


def _final_body(xa_ref, a0_ref, a1_ref, a2_ref, vd_ref, v3_ref,
                wo_ref, bo_ref, dx_ref, dvec_ref):
    o = jnp.dot(xa_ref[...], wo_ref[...], preferred_element_type=jnp.float32) + bo_ref[...]
    o1 = o[:, :D]
    o2 = o[:, D:2 * D]
    o3 = o[:, 2 * D:3 * D]
    dx_ref[...] = vd_ref[...] * o2 + o3
    for c, aref in enumerate((a0_ref, a1_ref, a2_ref)):
        dvec_ref[:, c, :] = v3_ref[:, c, :] * o1 + aref[...]


def _final_stage(xagg, a0, a1, a2, vd, v3, Wo, bo2):
    def full(shape):
        return pl.BlockSpec(shape, lambda i: tuple(0 for _ in shape))
    row = lambda w: pl.BlockSpec((BN3, w), lambda i: (i, 0))
    return pl.pallas_call(
        _final_body,
        grid=(N // BN3,),
        in_specs=[
            row(D), row(D), row(D), row(D), row(D),
            pl.BlockSpec((BN3, 3, D), lambda i: (i, 0, 0)),
            full((D, 3 * D)), full((1, 3 * D)),
        ],
        out_specs=[
            row(D),
            pl.BlockSpec((BN3, 3, D), lambda i: (i, 0, 0)),
        ],
        out_shape=[
            jax.ShapeDtypeStruct((N, D), jnp.float32),
            jax.ShapeDtypeStruct((N, 3, D), jnp.float32),
        ],
    )(xagg, a0, a1, a2, vd, v3, Wo, bo2)


def kernel(x, vec, edge_index, r_ij, f_ij, d_ij, g_ln, b_ln, Wq, bq, Wk, bk,
           Wv, bv, Wvec, Wdk, bdk, Wdv, bdv, Wo, bo):
    f32 = jnp.float32
    Wv_p = Wv.reshape(D, H, 3, DH).transpose(0, 2, 1, 3).reshape(D, 3 * D)
    bv_p = bv.reshape(H, 3, DH).transpose(1, 0, 2).reshape(3 * D)
    Wdv_p = Wdv.reshape(R, H, 3, DH).transpose(0, 2, 1, 3).reshape(R, 3 * D)
    bdv_p = bdv.reshape(H, 3, DH).transpose(1, 0, 2).reshape(3 * D)
    src = edge_index[0]
    dst = edge_index[1]
    cut2 = _cut_stage(r_ij.reshape(E // D, D)).reshape(E, 1)
    d0 = d_ij[:, 0:1]
    d1 = d_ij[:, 1:2]
    d2 = d_ij[:, 2:3]
    m = jnp.kron(jnp.eye(H, dtype=f32), jnp.ones((DH, DH), f32))
    zrows = jnp.zeros((NPAD, D), f32)

    q, kt, vd, v3 = _node_stage(
        x, vec, g_ln.reshape(1, D), b_ln.reshape(1, D),
        Wq, bq.reshape(1, D), Wvec)
    qd, g = _sc_gather(q, kt, src, dst)
    xe, vm0, vm1, vm2 = _edge_stage(
        qd, g, f_ij, cut2, d0, d1, d2,
        Wdk, bdk.reshape(1, D), Wdv_p, bdv_p.reshape(1, 3 * D),
        Wk, bk.reshape(1, D), Wv_p, bv_p.reshape(1, 3 * D), m)
    xagg, a0, a1, a2 = _sc_scatter(xe, vm0, vm1, vm2,
                                   dst.reshape(NS, NCHS, CHS), zrows)
    dx, dvec = _final_stage(xagg[:N], a0[:N], a1[:N], a2[:N], vd, v3,
                            Wo, bo.reshape(1, 3 * D))
    return dx, dvec


# speedup vs baseline: 39.5707x; 1.0028x over previous
import functools
import math

import jax
import jax.numpy as jnp
from jax import lax
from jax.experimental import pallas as pl
from jax.experimental.pallas import tpu as pltpu
from jax.experimental.pallas import tpu_sc as plsc

N = 10000
D = 128
H = 8
DH = 16
E = 160000
R = 64
CUT = 5.0

NC = 2
NS = 16
NW = NC * NS


BN1 = 1000


def _silu(a):
    return a / (1.0 + jnp.exp(-a))


def _pack_bf(x):
    u = jax.lax.bitcast_convert_type(x, jnp.uint32)
    r = (u + jnp.uint32(0x7FFF) + ((u >> 16) & jnp.uint32(1))) >> 16
    lo = r[:, :D // 2]
    hi = r[:, D // 2:]
    return jax.lax.bitcast_convert_type(lo | (hi << 16), jnp.int32)


def _unpack_bf(w):
    u = jax.lax.bitcast_convert_type(w, jnp.uint32)
    lo = jax.lax.bitcast_convert_type(u << 16, jnp.float32)
    hi = jax.lax.bitcast_convert_type(u & jnp.uint32(0xFFFF0000), jnp.float32)
    return jnp.concatenate([lo, hi], axis=1)


def _node_body(x_ref, vec_ref, r_ref, gln_ref, bln_ref, wq_ref, bq_ref,
               wvec_ref, q_ref, kt_ref, vd_ref, v3_ref, cut_ref):
    r = r_ref[...]
    cut_ref[...] = (0.5 * (jnp.cos(r * (math.pi / CUT)) + 1.0)
                    * (r < CUT).astype(jnp.float32))
    x = x_ref[...]
    mu = jnp.mean(x, axis=-1, keepdims=True)
    xc = x - mu
    var = jnp.mean(xc * xc, axis=-1, keepdims=True)
    xn = xc * lax.rsqrt(var + 1e-5) * gln_ref[...] + bln_ref[...]
    dot = lambda a, w: jnp.dot(a, w, preferred_element_type=jnp.float32)
    HD = D // 2
    q_ref[...] = dot(xn, wq_ref[...]) + bq_ref[...]
    kt_ref[:, 0:HD] = _pack_bf(xn)
    vd = jnp.zeros((x.shape[0], D), jnp.float32)
    for c in range(3):
        vc = vec_ref[:, c, :]
        vp = dot(vc, wvec_ref[...])
        vd = vd + vp[:, :D] * vp[:, D:2 * D]
        v3_ref[:, c, :] = vp[:, 2 * D:3 * D]
        kt_ref[:, HD + c * HD:HD + (c + 1) * HD] = _pack_bf(vc)
    vd_ref[...] = vd


EPAD = 163840


def _node_stage(x, vec, r_pad, g_ln2, b_ln2, Wq, bq2, Wvec):
    def full(shape):
        return pl.BlockSpec(shape, lambda i: tuple(0 for _ in shape))
    rb = EPAD // D // (N // BN1)
    return pl.pallas_call(
        _node_body,
        grid=(N // BN1,),
        in_specs=[
            pl.BlockSpec((BN1, D), lambda i: (i, 0)),
            pl.BlockSpec((BN1, 3, D), lambda i: (i, 0, 0)),
            pl.BlockSpec((rb, D), lambda i: (i, 0)),
            full((1, D)), full((1, D)),
            full((D, D)), full((1, D)),
            full((D, 3 * D)),
        ],
        out_specs=[
            pl.BlockSpec((BN1, D), lambda i: (i, 0)),
            pl.BlockSpec((BN1, 2 * D), lambda i: (i, 0)),
            pl.BlockSpec((BN1, D), lambda i: (i, 0)),
            pl.BlockSpec((BN1, 3, D), lambda i: (i, 0, 0)),
            pl.BlockSpec((rb, D), lambda i: (i, 0)),
        ],
        out_shape=[
            jax.ShapeDtypeStruct((N, D), jnp.float32),
            jax.ShapeDtypeStruct((N, 2 * D), jnp.int32),
            jax.ShapeDtypeStruct((N, D), jnp.float32),
            jax.ShapeDtypeStruct((N, 3, D), jnp.float32),
            jax.ShapeDtypeStruct((EPAD // D, D), jnp.float32),
        ],
    )(x, vec, r_pad, g_ln2, b_ln2, Wq, bq2, Wvec)


EPW = E // NW
CHG = 80
NCHF = EPW // CHG
RESG = EPW - NCHF * CHG


def _sc_gather(q, kt, src, dst):
    mesh = plsc.VectorSubcoreMesh(core_axis_name="c", subcore_axis_name="s")

    @functools.partial(
        pl.kernel,
        mesh=mesh,
        out_type=[jax.ShapeDtypeStruct((E, D), jnp.float32),
                  jax.ShapeDtypeStruct((E, 2 * D), jnp.int32)],
        scratch_types=[
            pltpu.VMEM((EPW,), jnp.int32),
            pltpu.VMEM((EPW,), jnp.int32),
            [pltpu.VMEM((CHG, D), jnp.float32)] * 2,
            [pltpu.VMEM((CHG, 2 * D), jnp.int32)] * 2,
            [pltpu.SemaphoreType.DMA] * 8,
        ],
    )
    def gk(q_hbm, kt_hbm, src_hbm, dst_hbm, qd_out, g_out,
           didx, sidx, qbuf, kbuf, sems):
        gq = sems[0:2]
        gk_ = sems[2:4]
        wq = sems[4:6]
        wk = sems[6:8]
        wid = lax.axis_index("s") * NC + lax.axis_index("c")
        base0 = wid * EPW
        pltpu.sync_copy(dst_hbm.at[pl.ds(base0, EPW)], didx)
        pltpu.sync_copy(src_hbm.at[pl.ds(base0, EPW)], sidx)

        def fire_gather(i, b, n=CHG):
            pltpu.async_copy(q_hbm.at[didx.at[pl.ds(i * CHG, n)]],
                             qbuf[b].at[pl.ds(0, n)], gq[b])
            pltpu.async_copy(kt_hbm.at[sidx.at[pl.ds(i * CHG, n)]],
                             kbuf[b].at[pl.ds(0, n)], gk_[b])

        def wait_gather(i, b, n=CHG):
            pltpu.make_async_copy(q_hbm.at[didx.at[pl.ds(i * CHG, n)]],
                                  qbuf[b].at[pl.ds(0, n)], gq[b]).wait()
            pltpu.make_async_copy(kt_hbm.at[sidx.at[pl.ds(i * CHG, n)]],
                                  kbuf[b].at[pl.ds(0, n)], gk_[b]).wait()

        def fire_write(i, b, n=CHG):
            base = base0 + i * CHG
            pltpu.async_copy(qbuf[b].at[pl.ds(0, n)],
                             qd_out.at[pl.ds(base, n)], wq[b])
            pltpu.async_copy(kbuf[b].at[pl.ds(0, n)],
                             g_out.at[pl.ds(base, n)], wk[b])

        def wait_write(i, b, n=CHG):
            base = base0 + i * CHG
            pltpu.make_async_copy(qbuf[b].at[pl.ds(0, n)],
                                  qd_out.at[pl.ds(base, n)], wq[b]).wait()
            pltpu.make_async_copy(kbuf[b].at[pl.ds(0, n)],
                                  g_out.at[pl.ds(base, n)], wk[b]).wait()

        fire_gather(0, 0)

        def pair(j, carry):
            i0 = 2 * j
            for b in range(2):
                i = i0 + b
                wait_gather(i, b)
                fire_write(i, b)
                bn = 1 - b

                @pl.when(i == 0)
                def _():
                    fire_gather(1, 1)

                @pl.when(jnp.logical_and(i >= 1, i + 1 < NCHF))
                def _(i=i, bn=bn):
                    wait_write(i - 1, bn)
                    fire_gather(i + 1, bn)
            return carry

        lax.fori_loop(0, NCHF // 2, pair, 0)
        wait_write(NCHF - 2, 0)
        wait_write(NCHF - 1, 1)
        if RESG:
            fire_gather(NCHF, 0, RESG)
            wait_gather(NCHF, 0, RESG)
            fire_write(NCHF, 0, RESG)
            wait_write(NCHF, 0, RESG)

    return gk(q, kt, src, dst)


BE = 1000


def _edge_body(qd_ref, g_ref, f_ref, cut_ref, d0_ref, d1_ref, d2_ref,
               wdk_ref, bdk_ref, wdv_ref, bdv_ref, wk_ref, bk_ref,
               wv_ref, bv_ref, m_ref,
               xe_ref, vm0_ref, vm1_ref, vm2_ref):
    dot = lambda a, w: jnp.dot(a, w, preferred_element_type=jnp.float32)
    f = f_ref[...]
    dk = _silu(dot(f, wdk_ref[...]) + bdk_ref[...])
    dvp = _silu(dot(f, wdv_ref[...]) + bdv_ref[...])
    HD = D // 2
    xn_s = _unpack_bf(g_ref[:, :HD])
    ks = dot(xn_s, wk_ref[...]) + bk_ref[...]
    vs = dot(xn_s, wv_ref[...]) + bv_ref[...]
    attn_pre = dot(qd_ref[...] * ks * dk, m_ref[...])
    a = _silu(attn_pre) * cut_ref[...]
    vjx = vs[:, :D] * dvp[:, :D]
    v1e = vs[:, D:2 * D] * dvp[:, D:2 * D]
    v2e = vs[:, 2 * D:3 * D] * dvp[:, 2 * D:3 * D]
    xe_ref[...] = vjx * a
    for c, (dref, oref) in enumerate(((d0_ref, vm0_ref), (d1_ref, vm1_ref),
                                      (d2_ref, vm2_ref))):
        vecc = _unpack_bf(g_ref[:, HD + c * HD:HD + (c + 1) * HD])
        oref[...] = vecc * v1e + v2e * dref[...]


def _edge_stage(qd, g, f_ij, cut2, d0, d1, d2, Wdk, bdk2, Wdv_p, bdv_p2,
                Wk, bk2, Wv_p, bv_p2, m):
    def full(shape):
        return pl.BlockSpec(shape, lambda i: tuple(0 for _ in shape))
    row = lambda w: pl.BlockSpec((BE, w), lambda i: (i, 0))
    return pl.pallas_call(
        _edge_body,
        grid=(E // BE,),
        in_specs=[
            row(D), row(2 * D), row(R), row(1), row(1), row(1), row(1),
            full((R, D)), full((1, D)),
            full((R, 3 * D)), full((1, 3 * D)),
            full((D, D)), full((1, D)),
            full((D, 3 * D)), full((1, 3 * D)),
            full((D, D)),
        ],
        out_specs=[row(D), row(D), row(D), row(D)],
        out_shape=[jax.ShapeDtypeStruct((E, D), jnp.float32)] * 4,
    )(qd, g, f_ij, cut2, d0, d1, d2, Wdk, bdk2, Wdv_p, bdv_p2,
      Wk, bk2, Wv_p, bv_p2, m)


EPT = E // NS
CHS = 80
NCHS = EPT // CHS
NPAD = 10240
NPT = NPAD // NS


def _sc_scatter(m0, m1, m2, m3, dst3, zrows):
    mesh = plsc.VectorSubcoreMesh(core_axis_name="c", subcore_axis_name="s")

    @functools.partial(
        pl.kernel,
        mesh=mesh,
        out_type=[jax.ShapeDtypeStruct((NPAD, D), jnp.float32)] * 4,
        scratch_types=[
            pltpu.VMEM((NCHS, CHS), jnp.int32),
            [pltpu.VMEM((CHS, D), jnp.float32)] * 2,
            pltpu.VMEM_SHARED((NPAD, D), jnp.float32),
            [pltpu.SemaphoreType.DMA] * 2,
        ],
    )
    def sk(m0_hbm, m1_hbm, m2_hbm, m3_hbm, dst_hbm, z_hbm,
           o0, o1, o2, o3, idx, rows, acc, sems):
        cid = lax.axis_index("c")
        sid = lax.axis_index("s")
        row0 = sid * NPT
        pltpu.sync_copy(dst_hbm.at[sid], idx)
        planes = ((m0_hbm, o0), (m1_hbm, o1), (m2_hbm, o2), (m3_hbm, o3))
        for plane, (m_hbm, o_hbm) in enumerate(planes):
            @pl.when(cid == (plane // 2))
            def _(m_hbm=m_hbm, o_hbm=o_hbm):
                pltpu.sync_copy(z_hbm.at[pl.ds(row0, NPT)],
                                acc.at[pl.ds(row0, NPT)])

                def fire_read(i, b):
                    base = sid * EPT + i * CHS
                    pltpu.async_copy(m_hbm.at[pl.ds(base, CHS)],
                                     rows[b], sems[b])

                def wait_read(i, b):
                    base = sid * EPT + i * CHS
                    pltpu.make_async_copy(m_hbm.at[pl.ds(base, CHS)],
                                          rows[b], sems[b]).wait()

                plsc.subcore_barrier()
                fire_read(0, 0)
                fire_read(1, 1)

                def pair(j, carry):
                    i0 = 2 * j
                    for b in range(2):
                        i = i0 + b
                        wait_read(i, b)
                        pltpu.sync_copy(rows[b], acc.at[idx.at[i]], add=True)

                        @pl.when(i + 2 < NCHS)
                        def _():
                            fire_read(i + 2, b)
                    return carry

                lax.fori_loop(0, NCHS // 2, pair, 0)
                if NCHS % 2:
                    last = NCHS - 1
                    wait_read(last, 0)
                    pltpu.sync_copy(rows[0], acc.at[idx.at[last]], add=True)
                plsc.subcore_barrier()
                pltpu.sync_copy(acc.at[pl.ds(row0, NPT)],
                                o_hbm.at[pl.ds(row0, NPT)])
                plsc.subcore_barrier()

    return sk(m0, m1, m2, m3, dst3, zrows)


BN3 = 2000


def _final_body(xa_ref, a0_ref, a1_ref, a2_ref, vd_ref, v3_ref,
                wo_ref, bo_ref, dx_ref, dvec_ref):
    o = jnp.dot(xa_ref[...], wo_ref[...], preferred_element_type=jnp.float32) + bo_ref[...]
    o1 = o[:, :D]
    o2 = o[:, D:2 * D]
    o3 = o[:, 2 * D:3 * D]
    dx_ref[...] = vd_ref[...] * o2 + o3
    for c, aref in enumerate((a0_ref, a1_ref, a2_ref)):
        dvec_ref[:, c, :] = v3_ref[:, c, :] * o1 + aref[...]


def _final_stage(xagg, a0, a1, a2, vd, v3, Wo, bo2):
    def full(shape):
        return pl.BlockSpec(shape, lambda i: tuple(0 for _ in shape))
    row = lambda w: pl.BlockSpec((BN3, w), lambda i: (i, 0))
    return pl.pallas_call(
        _final_body,
        grid=(N // BN3,),
        in_specs=[
            row(D), row(D), row(D), row(D), row(D),
            pl.BlockSpec((BN3, 3, D), lambda i: (i, 0, 0)),
            full((D, 3 * D)), full((1, 3 * D)),
        ],
        out_specs=[
            row(D),
            pl.BlockSpec((BN3, 3, D), lambda i: (i, 0, 0)),
        ],
        out_shape=[
            jax.ShapeDtypeStruct((N, D), jnp.float32),
            jax.ShapeDtypeStruct((N, 3, D), jnp.float32),
        ],
    )(xagg, a0, a1, a2, vd, v3, Wo, bo2)


def kernel(x, vec, edge_index, r_ij, f_ij, d_ij, g_ln, b_ln, Wq, bq, Wk, bk,
           Wv, bv, Wvec, Wdk, bdk, Wdv, bdv, Wo, bo):
    f32 = jnp.float32
    Wv_p = Wv.reshape(D, H, 3, DH).transpose(0, 2, 1, 3).reshape(D, 3 * D)
    bv_p = bv.reshape(H, 3, DH).transpose(1, 0, 2).reshape(3 * D)
    Wdv_p = Wdv.reshape(R, H, 3, DH).transpose(0, 2, 1, 3).reshape(R, 3 * D)
    bdv_p = bdv.reshape(H, 3, DH).transpose(1, 0, 2).reshape(3 * D)
    src = edge_index[0]
    dst = edge_index[1]
    r_pad = jnp.pad(r_ij.reshape(E // D, D), ((0, (EPAD - E) // D), (0, 0)))
    d0 = d_ij[:, 0:1]
    d1 = d_ij[:, 1:2]
    d2 = d_ij[:, 2:3]
    m = jnp.kron(jnp.eye(H, dtype=f32), jnp.ones((DH, DH), f32))
    zrows = jnp.zeros((NPAD, D), f32)

    q, kt, vd, v3, cut_pad = _node_stage(
        x, vec, r_pad, g_ln.reshape(1, D), b_ln.reshape(1, D),
        Wq, bq.reshape(1, D), Wvec)
    cut2 = cut_pad.reshape(EPAD, 1)[:E]
    qd, g = _sc_gather(q, kt, src, dst)
    xe, vm0, vm1, vm2 = _edge_stage(
        qd, g, f_ij, cut2, d0, d1, d2,
        Wdk, bdk.reshape(1, D), Wdv_p, bdv_p.reshape(1, 3 * D),
        Wk, bk.reshape(1, D), Wv_p, bv_p.reshape(1, 3 * D), m)
    xagg, a0, a1, a2 = _sc_scatter(xe, vm0, vm1, vm2,
                                   dst.reshape(NS, NCHS, CHS), zrows)
    dx, dvec = _final_stage(xagg[:N], a0[:N], a1[:N], a2[:N], vd, v3,
                            Wo, bo.reshape(1, 3 * D))
    return dx, dvec
